# baseline (speedup 1.0000x reference)
import functools

import jax
import jax.numpy as jnp
import numpy as np
from jax import lax
from jax.experimental import pallas as pl
from jax.experimental.pallas import tpu as pltpu
from jax.experimental.pallas import tpu_sc as plsc

_N = 10000
_E = 160000
_T = 8
_F = 26
_H = 64
_HOR = 4
_OUTF = 26

_NB = 5
_BN = _N // _NB

_NC = 2
_NS = 16
_NW = _NC * _NS
_CH = 64
_PER_TILE = 5120
_EPAD = _PER_TILE * _NW
_NCH = _PER_TILE // _CH
_NPAD = 10016
_RPT = _NPAD // _NS

_PLO = np.array([32 * g + k for g in range(_H // 32) for k in range(16)],
                np.int32)
_PHI = _PLO + 16


def _lk(v, s=0.01):
    return jnp.where(v >= 0, v, s * v)


def _dot(a, b):
    return jnp.dot(a, b, preferred_element_type=jnp.float32)


def _enc_body(x_ref, e_ref, wx_ref, wu_ref, b_ref, wsk_ref, w1_ref, b1_ref,
              w2_ref, b2_ref, emb_ref, o_ref):
    xb = x_ref[0]
    eb = e_ref[0]
    h = _lk(_dot(xb, wx_ref[...]) + eb * wu_ref[...] + b_ref[...])
    h = h + _dot(xb, wsk_ref[...])
    h = _lk(_dot(h, w1_ref[...]) + b1_ref[...])
    h = _lk(_dot(h, w2_ref[...]) + b2_ref[...])
    o_ref[0] = h + emb_ref[...]


def _encoder(x3, ex, wx, wu_eff, b, wsk, w1, b1, w2, b2, emb):
    def wspec(shape):
        return pl.BlockSpec(shape, lambda t, nb: (0,) * len(shape))
    return pl.pallas_call(
        _enc_body,
        grid=(_T, _NB),
        in_specs=[
            pl.BlockSpec((1, _BN, _F), lambda t, nb: (t, nb, 0)),
            pl.BlockSpec((1, _BN, 1), lambda t, nb: (t, nb, 0)),
            wspec((_F, _H)), wspec((1, _H)), wspec((1, _H)), wspec((_F, _H)),
            wspec((_H, 2 * _H)), wspec((1, 2 * _H)),
            wspec((2 * _H, _H)), wspec((1, _H)),
            pl.BlockSpec((_BN, _H), lambda t, nb: (nb, 0)),
        ],
        out_specs=pl.BlockSpec((1, _BN, _H), lambda t, nb: (t, nb, 0)),
        out_shape=jax.ShapeDtypeStruct((_T, _N, _H), jnp.float32),
    )(x3, ex, wx, wu_eff, b, wsk, w1, b1, w2, b2, emb)


def _bfpack(xlo, xhi):
    ulo = jax.lax.bitcast_convert_type(xlo, jnp.uint32)
    uhi = jax.lax.bitcast_convert_type(xhi, jnp.uint32)
    one = jnp.uint32(1)
    half = jnp.uint32(0x7FFF)
    rlo = (ulo + half + ((ulo >> 16) & one)) >> 16
    rhi = (uhi + half + ((uhi >> 16) & one)) >> 16
    return jax.lax.bitcast_convert_type(rlo | (rhi << 16), jnp.int32)


def _tmp_body_mk(has_prev):
    def body(*refs):
        if has_prev:
            (h1_ref, h0_ref, po_ref, wt1_ref, wt0_ref, bt_ref, walo_ref,
             wahi_ref, wblo_ref, wbhi_ref, wvlo_ref, wvhi_ref, wu_ref,
             ws_ref, bsk_ref, ah_ref, bv_ref, uh_ref, on_ref) = refs
        else:
            (h1_ref, h0_ref, wt1_ref, wt0_ref, bt_ref, walo_ref,
             wahi_ref, wblo_ref, wbhi_ref, wvlo_ref, wvhi_ref, wu_ref,
             ws_ref, bsk_ref, ah_ref, bv_ref, uh_ref, on_ref) = refs
        hc = _lk(_dot(h1_ref[0], wt1_ref[...]) + _dot(h0_ref[0], wt0_ref[...])
                 + bt_ref[...])
        ah_ref[...] = _bfpack(_dot(hc, walo_ref[...]), _dot(hc, wahi_ref[...]))
        bv_ref[...] = jnp.concatenate(
            [_bfpack(_dot(hc, wblo_ref[...]), _dot(hc, wbhi_ref[...])),
             _bfpack(_dot(hc, wvlo_ref[...]), _dot(hc, wvhi_ref[...]))],
            axis=1)
        uh_ref[0] = _dot(hc, wu_ref[...])
        on = _dot(hc, ws_ref[...]) + bsk_ref[...]
        if has_prev:
            on = on + po_ref[0]
        on_ref[0] = on
    return body


def _temporal(h, po, lp, d, tl, tprev):
    def wspec(shape):
        return pl.BlockSpec(shape, lambda t, nb: (0,) * len(shape))
    hspec = lambda off: pl.BlockSpec((1, _BN, _H), lambda t, nb: (t + off, nb, 0))
    tab_spec = lambda wdt: pl.BlockSpec((_BN, wdt), lambda t, nb: (t * _NB + nb, 0))
    tab_shape = lambda wdt: jax.ShapeDtypeStruct((tl * _N + 16, wdt), jnp.int32)
    seq_spec = pl.BlockSpec((1, _BN, _H), lambda t, nb: (t, nb, 0))
    seq_shape = jax.ShapeDtypeStruct((tl, _N, _H), jnp.float32)
    hw = _H // 2
    has_prev = po is not None
    po_spec = ([pl.BlockSpec((1, _BN, _H),
                             lambda t, nb: (t + (tprev - tl), nb, 0))]
               if has_prev else [])
    po_arg = [po] if has_prev else []
    return pl.pallas_call(
        _tmp_body_mk(has_prev),
        grid=(tl, _NB),
        in_specs=[
            hspec(d), hspec(0), *po_spec,
            wspec((_H, _H)), wspec((_H, _H)), wspec((1, _H)),
            wspec((_H, hw)), wspec((_H, hw)), wspec((_H, hw)),
            wspec((_H, hw)), wspec((_H, hw)), wspec((_H, hw)),
            wspec((_H, _H)), wspec((_H, _H)), wspec((1, _H)),
        ],
        out_specs=[tab_spec(hw), tab_spec(_H), seq_spec, seq_spec],
        out_shape=[tab_shape(hw), tab_shape(_H), seq_shape, seq_shape],
    )(h, h, *po_arg, lp["Wt1"], lp["Wt0"], lp["bt"].reshape(1, _H),
      lp["A"][:, _PLO], lp["A"][:, _PHI], lp["B"][:, _PLO], lp["B"][:, _PHI],
      lp["V"][:, _PLO], lp["V"][:, _PHI], lp["U"], lp["Ws"],
      lp["bskip"].reshape(1, _H))


def _edge_sc(tl):
    mesh = plsc.VectorSubcoreMesh(core_axis_name="c", subcore_axis_name="s")

    @functools.partial(
        pl.kernel,
        out_type=jax.ShapeDtypeStruct((_NC, tl, _NPAD, 128), jnp.float32),
        mesh=mesh,
        compiler_params=pltpu.CompilerParams(use_tc_tiling_on_sc=False),
        scratch_types=[
            pltpu.VMEM((_NCH, _CH), jnp.int32),
            pltpu.VMEM((_NCH, _CH), jnp.int32),
            pltpu.VMEM((_NCH, _CH), jnp.int32),
            pltpu.VMEM((_CH, _H // 2), jnp.int32),
            pltpu.VMEM((_CH, _H), jnp.int32),
            pltpu.VMEM((_CH, _H // 2), jnp.int32),
            pltpu.VMEM((_CH, _H), jnp.int32),
            pltpu.VMEM((_CH, 128), jnp.float32),
            pltpu.VMEM((_CH, 128), jnp.float32),
            pltpu.VMEM((16, 128), jnp.float32),
            pltpu.VMEM_SHARED((_NPAD, 128), jnp.float32),
            pltpu.SemaphoreType.DMA,
            pltpu.SemaphoreType.DMA,
            pltpu.SemaphoreType.DMA,
            pltpu.SemaphoreType.DMA,
        ],
    )
    def k(ah, bv, srcr, dstr, out, srco, dsti, dsto, ar0, bv0,
          ar1, bv1, ob0, ob1, zbuf, accum, sg0, sg1, ss0, ss1):
        c = lax.axis_index("c")
        s = lax.axis_index("s")
        w = c * _NS + s
        row0 = s * _RPT
        bufs = ((ar0, bv0, ob0, sg0, ss0),
                (ar1, bv1, ob1, sg1, ss1))

        pltpu.sync_copy(srcr.at[w], srco)
        pltpu.sync_copy(dstr.at[w], dsti)
        pltpu.sync_copy(dstr.at[w], dsto)

        def zrow(r, carry):
            for j in range(8):
                zbuf[r, pl.ds(16 * j, 16)] = jnp.zeros((16,), jnp.float32)
            return carry
        lax.fori_loop(0, 16, zrow, 0)

        def issue_gather(ck, b):
            arb, bvb, sgb = bufs[b][0], bufs[b][1], bufs[b][3]
            pltpu.async_copy(ah.at[dsto.at[ck]], arb, sgb)
            pltpu.async_copy(bv.at[srco.at[ck]], bvb, sgb)

        def wait_gather(b):
            arb, bvb, sgb = bufs[b][0], bufs[b][1], bufs[b][3]
            pltpu.make_async_copy(ah.at[dsto.at[0]], arb, sgb).wait()
            pltpu.make_async_copy(bv.at[srco.at[0]], bvb, sgb).wait()

        def wait_scatter(b):
            obb, ssb = bufs[b][2], bufs[b][4]
            pltpu.make_async_copy(obb, accum.at[dsti.at[0]], ssb).wait()

        def tt_body(tt, carry):
            def zcp(kk, cc):
                pltpu.sync_copy(zbuf, accum.at[pl.ds(row0 + kk * 16, 16)])
                return cc
            lax.fori_loop(0, _RPT // 16, zcp, 0)
            rem = _RPT % 16
            if rem:
                pltpu.sync_copy(zbuf.at[pl.ds(0, rem)],
                                accum.at[pl.ds(row0 + _RPT - rem, rem)])
            plsc.subcore_barrier()

            def off(ck2, c2):
                for j in range(_CH // 16):
                    sl = pl.ds(16 * j, 16)
                    srco[ck2, sl] = srco[ck2, sl] + _N
                    dsto[ck2, sl] = dsto[ck2, sl] + _N
                return c2

            @pl.when(tt > 0)
            def _():
                lax.fori_loop(0, _NCH, off, 0)

            issue_gather(0, 0)

            def pair(i, c2):
                for b in (0, 1):
                    ck = 2 * i + b
                    if b == 0:
                        issue_gather(ck + 1, 1)
                    else:
                        @pl.when(i < _NCH // 2 - 1)
                        def _():
                            issue_gather(ck + 1, 0)
                    wait_gather(b)

                    @pl.when(i > 0)
                    def _():
                        wait_scatter(b)

                    arb, bvb, obb, _, ssb = bufs[b]

                    def _unp(word):
                        lo = jax.lax.bitcast_convert_type(word << 16,
                                                          jnp.float32)
                        hi = jax.lax.bitcast_convert_type(word, jnp.float32)
                        return lo, hi

                    @plsc.parallel_loop(0, _CH, 1, unroll=4)
                    def ebody(e2):
                        for j in range(_H // 32):
                            sl = pl.ds(16 * j, 16)
                            a0, a1 = _unp(arb[e2, sl])
                            b0, b1 = _unp(bvb[e2, sl])
                            v0, v1 = _unp(bvb[e2, pl.ds(_H // 2 + 16 * j, 16)])
                            g0 = 1.0 / (1.0 + jnp.exp(-(a0 + b0)))
                            g1 = 1.0 / (1.0 + jnp.exp(-(a1 + b1)))
                            obb[e2, pl.ds(32 * j, 16)] = g0 * v0
                            obb[e2, pl.ds(32 * j + 16, 16)] = g1 * v1
                            obb[e2, pl.ds(_H + 32 * j, 16)] = g0
                            obb[e2, pl.ds(_H + 32 * j + 16, 16)] = g1

                    pltpu.async_copy(obb, accum.at[dsti.at[ck]], ssb,
                                     add=True)
                return c2
            lax.fori_loop(0, _NCH // 2, pair, 0)
            wait_scatter(0)
            wait_scatter(1)
            plsc.subcore_barrier()

            pltpu.sync_copy(accum.at[pl.ds(row0, _RPT)],
                            out.at[c, tt, pl.ds(row0, _RPT)])
            plsc.subcore_barrier()
            return carry
        lax.fori_loop(0, tl, tt_body, 0)

    return k


def _cmb_body(acc_ref, uh_ref, res_ref, bs_ref, o_ref):
    sm = (acc_ref[0, 0].astype(jnp.float32)
          + acc_ref[1, 0].astype(jnp.float32))
    num = sm[:, :_H]
    den = sm[:, _H:]
    g = uh_ref[0] + num / (den + 1e-6) + bs_ref[...]
    o_ref[0] = jnp.where(g >= 0, g, 0.1 * g) + res_ref[0]


def _combine(acc, uh, res, bs, tl, tin):
    return pl.pallas_call(
        _cmb_body,
        grid=(tl, _NB),
        in_specs=[
            pl.BlockSpec((_NC, 1, _BN, 128), lambda t, nb: (0, t, nb, 0)),
            pl.BlockSpec((1, _BN, _H), lambda t, nb: (t, nb, 0)),
            pl.BlockSpec((1, _BN, _H), lambda t, nb: (t + (tin - tl), nb, 0)),
            pl.BlockSpec((1, _H), lambda t, nb: (0, 0)),
        ],
        out_specs=pl.BlockSpec((1, _BN, _H), lambda t, nb: (t, nb, 0)),
        out_shape=jax.ShapeDtypeStruct((tl, _N, _H), jnp.float32),
    )(acc, uh, res, bs.reshape(1, _H))


def _dec_body(h_ref, o_ref, x_ref, r1_ref, rb1_ref, r2_ref, rb2_ref, r3_ref,
              rb3_ref, r4_ref, rb4_ref, y_ref):
    z = h_ref[0] + o_ref[0]
    m = _lk(_dot(z, r1_ref[...]) + rb1_ref[...])
    m = _lk(_dot(m, r2_ref[...]) + rb2_ref[...])
    m = _lk(_dot(m, r3_ref[...]) + rb3_ref[...])
    m = _dot(m, r4_ref[...]) + rb4_ref[...]
    xb = x_ref[0]
    y_ref[...] = m + jnp.concatenate([xb, xb, xb, xb], axis=1)


def _decoder(h, out_acc, x3, p, tl):
    def wspec(shape):
        return pl.BlockSpec(shape, lambda nb: (0,) * len(shape))
    last = lambda nb: (tl - 1, nb, 0)
    return pl.pallas_call(
        _dec_body,
        grid=(_NB,),
        in_specs=[
            pl.BlockSpec((1, _BN, _H), last),
            pl.BlockSpec((1, _BN, _H), last),
            pl.BlockSpec((1, _BN, _F), lambda nb: (_T - 1, nb, 0)),
            wspec((_H, 2 * _H)), wspec((1, 2 * _H)),
            wspec((2 * _H, 2 * _H)), wspec((1, 2 * _H)),
            wspec((2 * _H, 2 * _H)), wspec((1, 2 * _H)),
            wspec((2 * _H, _HOR * _OUTF)), wspec((1, _HOR * _OUTF)),
        ],
        out_specs=pl.BlockSpec((_BN, _HOR * _OUTF), lambda nb: (nb, 0)),
        out_shape=jax.ShapeDtypeStruct((_N, _HOR * _OUTF), jnp.float32),
    )(h, out_acc, x3,
      p["R1"], p["rb1"].reshape(1, -1), p["R2"], p["rb2"].reshape(1, -1),
      p["R3"], p["rb3"].reshape(1, -1), p["R4"], p["rb4"].reshape(1, -1))


def kernel(x, exog, params, edge_index):
    p = params
    x3 = x[0]
    ex = exog[0, :, :, 1:2]
    wu_eff = p["cond_Wu"][0:1] / 365.0 + p["cond_Wu"][1:2]

    h = _encoder(x3, ex, p["cond_Wx"], wu_eff, p["cond_b"].reshape(1, _H),
                 p["cond_skip"], p["enc_W1"], p["enc_b1"].reshape(1, 2 * _H),
                 p["enc_W2"], p["enc_b2"].reshape(1, _H), p["node_emb"])

    pad = _EPAD - _E
    pad_dst = _N + (jnp.arange(pad, dtype=jnp.int32) % (_NPAD - _N))
    order = jnp.argsort(edge_index[0])
    srcp = jnp.concatenate([edge_index[0][order], jnp.zeros((pad,), jnp.int32)])
    dstp = jnp.concatenate([edge_index[1][order], pad_dst])
    srcp = srcp.reshape(_NW, _NCH, _CH)
    dstp = dstp.reshape(_NW, _NCH, _CH)

    out_acc = None
    tin = _T
    tprev = _T - 1
    for l in range(2):
        lp = p["layers"][l]
        d = 2 ** (l % 2)
        tl = tin - d
        ah, bvt, uh, out_acc = _temporal(h, out_acc, lp, d, tl, tprev)
        acc = _edge_sc(tl)(ah, bvt, srcp, dstp)
        h = _combine(acc, uh, h, lp["bs"], tl, tin)
        tin = tl
        tprev = tl

    y = _decoder(h, out_acc, x3, p, tin)
    res = y.reshape(_N, _HOR, _OUTF).transpose(1, 0, 2)[None]
    return res


# speedup vs baseline: 1.1212x; 1.1212x over previous
import functools

import jax
import jax.numpy as jnp
import numpy as np
from jax import lax
from jax.experimental import pallas as pl
from jax.experimental.pallas import tpu as pltpu
from jax.experimental.pallas import tpu_sc as plsc

_N = 10000
_E = 160000
_T = 8
_F = 26
_H = 64
_HOR = 4
_OUTF = 26

_NB = 5
_BN = _N // _NB

_NC = 2
_NS = 16
_NW = _NC * _NS
_CH = 64
_PER_TILE = 5120
_EPAD = _PER_TILE * _NW
_NCH = _PER_TILE // _CH
_NPAD = 10016
_RPT = _NPAD // _NS

_PLO = np.array([32 * g + k for g in range(_H // 32) for k in range(16)],
                np.int32)
_PHI = _PLO + 16


def _lk(v, s=0.01):
    return jnp.where(v >= 0, v, s * v)


def _dot(a, b):
    return jnp.dot(a, b, preferred_element_type=jnp.float32)


def _enc_body(x_ref, e_ref, wx_ref, wu_ref, b_ref, wsk_ref, w1_ref, b1_ref,
              w2_ref, b2_ref, emb_ref, o_ref):
    xb = x_ref[0]
    eb = e_ref[0]
    h = _lk(_dot(xb, wx_ref[...]) + eb * wu_ref[...] + b_ref[...])
    h = h + _dot(xb, wsk_ref[...])
    h = _lk(_dot(h, w1_ref[...]) + b1_ref[...])
    h = _lk(_dot(h, w2_ref[...]) + b2_ref[...])
    o_ref[0] = h + emb_ref[...]


def _encoder(x3, ex, wx, wu_eff, b, wsk, w1, b1, w2, b2, emb):
    def wspec(shape):
        return pl.BlockSpec(shape, lambda t, nb: (0,) * len(shape))
    return pl.pallas_call(
        _enc_body,
        grid=(_T, _NB),
        in_specs=[
            pl.BlockSpec((1, _BN, _F), lambda t, nb: (t, nb, 0)),
            pl.BlockSpec((1, _BN, 1), lambda t, nb: (t, nb, 0)),
            wspec((_F, _H)), wspec((1, _H)), wspec((1, _H)), wspec((_F, _H)),
            wspec((_H, 2 * _H)), wspec((1, 2 * _H)),
            wspec((2 * _H, _H)), wspec((1, _H)),
            pl.BlockSpec((_BN, _H), lambda t, nb: (nb, 0)),
        ],
        out_specs=pl.BlockSpec((1, _BN, _H), lambda t, nb: (t, nb, 0)),
        out_shape=jax.ShapeDtypeStruct((_T, _N, _H), jnp.float32),
    )(x3, ex, wx, wu_eff, b, wsk, w1, b1, w2, b2, emb)


def _bfpack(xlo, xhi):
    ulo = jax.lax.bitcast_convert_type(xlo, jnp.uint32)
    uhi = jax.lax.bitcast_convert_type(xhi, jnp.uint32)
    one = jnp.uint32(1)
    half = jnp.uint32(0x7FFF)
    rlo = (ulo + half + ((ulo >> 16) & one)) >> 16
    rhi = (uhi + half + ((uhi >> 16) & one)) >> 16
    return jax.lax.bitcast_convert_type(rlo | (rhi << 16), jnp.int32)


def _tmp_body_mk(has_prev):
    def body(*refs):
        if has_prev:
            (h1_ref, h0_ref, po_ref, wt1_ref, wt0_ref, bt_ref, walo_ref,
             wahi_ref, wblo_ref, wbhi_ref, wvlo_ref, wvhi_ref, wu_ref,
             ws_ref, bsk_ref, ah_ref, bv_ref, uh_ref, on_ref) = refs
        else:
            (h1_ref, h0_ref, wt1_ref, wt0_ref, bt_ref, walo_ref,
             wahi_ref, wblo_ref, wbhi_ref, wvlo_ref, wvhi_ref, wu_ref,
             ws_ref, bsk_ref, ah_ref, bv_ref, uh_ref, on_ref) = refs
        hc = _lk(_dot(h1_ref[0], wt1_ref[...]) + _dot(h0_ref[0], wt0_ref[...])
                 + bt_ref[...])
        ah_ref[...] = _bfpack(_dot(hc, walo_ref[...]), _dot(hc, wahi_ref[...]))
        bv_ref[...] = jnp.concatenate(
            [_bfpack(_dot(hc, wblo_ref[...]), _dot(hc, wbhi_ref[...])),
             _bfpack(_dot(hc, wvlo_ref[...]), _dot(hc, wvhi_ref[...]))],
            axis=1)
        uh_ref[0] = _dot(hc, wu_ref[...])
        on = _dot(hc, ws_ref[...]) + bsk_ref[...]
        if has_prev:
            on = on + po_ref[0]
        on_ref[0] = on
    return body


def _temporal(h, po, lp, d, tl, tprev):
    def wspec(shape):
        return pl.BlockSpec(shape, lambda t, nb: (0,) * len(shape))
    hspec = lambda off: pl.BlockSpec((1, _BN, _H), lambda t, nb: (t + off, nb, 0))
    tab_spec = lambda wdt: pl.BlockSpec((_BN, wdt), lambda t, nb: (t * _NB + nb, 0))
    tab_shape = lambda wdt: jax.ShapeDtypeStruct((tl * _N + 16, wdt), jnp.int32)
    seq_spec = pl.BlockSpec((1, _BN, _H), lambda t, nb: (t, nb, 0))
    seq_shape = jax.ShapeDtypeStruct((tl, _N, _H), jnp.float32)
    hw = _H // 2
    has_prev = po is not None
    po_spec = ([pl.BlockSpec((1, _BN, _H),
                             lambda t, nb: (t + (tprev - tl), nb, 0))]
               if has_prev else [])
    po_arg = [po] if has_prev else []
    return pl.pallas_call(
        _tmp_body_mk(has_prev),
        grid=(tl, _NB),
        in_specs=[
            hspec(d), hspec(0), *po_spec,
            wspec((_H, _H)), wspec((_H, _H)), wspec((1, _H)),
            wspec((_H, hw)), wspec((_H, hw)), wspec((_H, hw)),
            wspec((_H, hw)), wspec((_H, hw)), wspec((_H, hw)),
            wspec((_H, _H)), wspec((_H, _H)), wspec((1, _H)),
        ],
        out_specs=[tab_spec(hw), tab_spec(_H), seq_spec, seq_spec],
        out_shape=[tab_shape(hw), tab_shape(_H), seq_shape, seq_shape],
    )(h, h, *po_arg, lp["Wt1"], lp["Wt0"], lp["bt"].reshape(1, _H),
      lp["A"][:, _PLO], lp["A"][:, _PHI], lp["B"][:, _PLO], lp["B"][:, _PHI],
      lp["V"][:, _PLO], lp["V"][:, _PHI], lp["U"], lp["Ws"],
      lp["bskip"].reshape(1, _H))


def _edge_sc(tl):
    mesh = plsc.VectorSubcoreMesh(core_axis_name="c", subcore_axis_name="s")

    @functools.partial(
        pl.kernel,
        out_type=jax.ShapeDtypeStruct((_NC, tl, _NPAD, 128), jnp.float32),
        mesh=mesh,
        compiler_params=pltpu.CompilerParams(use_tc_tiling_on_sc=False),
        scratch_types=[
            pltpu.VMEM((_NCH, _CH), jnp.int32),
            pltpu.VMEM((_NCH, _CH), jnp.int32),
            pltpu.VMEM((_NCH, _CH), jnp.int32),
            pltpu.VMEM((_CH, _H // 2), jnp.int32),
            pltpu.VMEM((_CH, _H), jnp.int32),
            pltpu.VMEM((_CH, _H // 2), jnp.int32),
            pltpu.VMEM((_CH, _H), jnp.int32),
            pltpu.VMEM((_CH, 128), jnp.float32),
            pltpu.VMEM((_CH, 128), jnp.float32),
            pltpu.VMEM((16, 128), jnp.float32),
            pltpu.VMEM_SHARED((_NPAD, 128), jnp.float32),
            pltpu.SemaphoreType.DMA,
            pltpu.SemaphoreType.DMA,
            pltpu.SemaphoreType.DMA,
            pltpu.SemaphoreType.DMA,
        ],
    )
    def k(ah, bv, srcr, dstr, out, srco, dsti, dsto, ar0, bv0,
          ar1, bv1, ob0, ob1, zbuf, accum, sg0, sg1, ss0, ss1):
        c = lax.axis_index("c")
        s = lax.axis_index("s")
        w = c * _NS + s
        row0 = s * _RPT
        bufs = ((ar0, bv0, ob0, sg0, ss0),
                (ar1, bv1, ob1, sg1, ss1))

        pltpu.sync_copy(srcr.at[w], srco)
        pltpu.sync_copy(dstr.at[w], dsti)
        pltpu.sync_copy(dstr.at[w], dsto)

        def zrow(r, carry):
            for j in range(8):
                zbuf[r, pl.ds(16 * j, 16)] = jnp.zeros((16,), jnp.float32)
            return carry
        lax.fori_loop(0, 16, zrow, 0)

        def issue_gather(ck, b):
            arb, bvb, sgb = bufs[b][0], bufs[b][1], bufs[b][3]
            pltpu.async_copy(ah.at[dsto.at[ck]], arb, sgb)
            pltpu.async_copy(bv.at[srco.at[ck]], bvb, sgb)

        def wait_gather(b):
            arb, bvb, sgb = bufs[b][0], bufs[b][1], bufs[b][3]
            pltpu.make_async_copy(ah.at[dsto.at[0]], arb, sgb).wait()
            pltpu.make_async_copy(bv.at[srco.at[0]], bvb, sgb).wait()

        def wait_scatter(b):
            obb, ssb = bufs[b][2], bufs[b][4]
            pltpu.make_async_copy(obb, accum.at[dsti.at[0]], ssb).wait()

        def tt_body(tt, carry):
            def zcp(kk, cc):
                pltpu.sync_copy(zbuf, accum.at[pl.ds(row0 + kk * 16, 16)])
                return cc
            lax.fori_loop(0, _RPT // 16, zcp, 0)
            rem = _RPT % 16
            if rem:
                pltpu.sync_copy(zbuf.at[pl.ds(0, rem)],
                                accum.at[pl.ds(row0 + _RPT - rem, rem)])
            plsc.subcore_barrier()

            def off(ck2, c2):
                for j in range(_CH // 16):
                    sl = pl.ds(16 * j, 16)
                    srco[ck2, sl] = srco[ck2, sl] + _N
                    dsto[ck2, sl] = dsto[ck2, sl] + _N
                return c2

            @pl.when(tt > 0)
            def _():
                lax.fori_loop(0, _NCH, off, 0)

            issue_gather(0, 0)

            def pair(i, c2):
                for b in (0, 1):
                    ck = 2 * i + b
                    if b == 0:
                        issue_gather(ck + 1, 1)
                    else:
                        @pl.when(i < _NCH // 2 - 1)
                        def _():
                            issue_gather(ck + 1, 0)
                    wait_gather(b)

                    @pl.when(i > 0)
                    def _():
                        wait_scatter(b)

                    arb, bvb, obb, _, ssb = bufs[b]

                    def _unp(word):
                        lo = jax.lax.bitcast_convert_type(word << 16,
                                                          jnp.float32)
                        hi = jax.lax.bitcast_convert_type(word, jnp.float32)
                        return lo, hi

                    @plsc.parallel_loop(0, _CH, 1, unroll=4)
                    def ebody(e2):
                        for j in range(_H // 32):
                            sl = pl.ds(16 * j, 16)
                            a0, a1 = _unp(arb[e2, sl])
                            b0, b1 = _unp(bvb[e2, sl])
                            v0, v1 = _unp(bvb[e2, pl.ds(_H // 2 + 16 * j, 16)])
                            g0 = 1.0 / (1.0 + jnp.exp(-(a0 + b0)))
                            g1 = 1.0 / (1.0 + jnp.exp(-(a1 + b1)))
                            obb[e2, pl.ds(32 * j, 16)] = g0 * v0
                            obb[e2, pl.ds(32 * j + 16, 16)] = g1 * v1
                            obb[e2, pl.ds(_H + 32 * j, 16)] = g0
                            obb[e2, pl.ds(_H + 32 * j + 16, 16)] = g1

                    pltpu.async_copy(obb, accum.at[dsti.at[ck]], ssb,
                                     add=True)
                return c2
            lax.fori_loop(0, _NCH // 2, pair, 0)
            wait_scatter(0)
            wait_scatter(1)
            plsc.subcore_barrier()

            pltpu.sync_copy(accum.at[pl.ds(row0, _RPT)],
                            out.at[c, tt, pl.ds(row0, _RPT)])
            plsc.subcore_barrier()
            return carry
        lax.fori_loop(0, tl, tt_body, 0)

    return k


def _cmb_body(acc_ref, uh_ref, res_ref, bs_ref, o_ref):
    sm = (acc_ref[0, 0].astype(jnp.float32)
          + acc_ref[1, 0].astype(jnp.float32))
    num = sm[:, :_H]
    den = sm[:, _H:]
    g = uh_ref[0] + num / (den + 1e-6) + bs_ref[...]
    o_ref[0] = jnp.where(g >= 0, g, 0.1 * g) + res_ref[0]


def _combine(acc, uh, res, bs, tl, tin):
    return pl.pallas_call(
        _cmb_body,
        grid=(tl, _NB),
        in_specs=[
            pl.BlockSpec((_NC, 1, _BN, 128), lambda t, nb: (0, t, nb, 0)),
            pl.BlockSpec((1, _BN, _H), lambda t, nb: (t, nb, 0)),
            pl.BlockSpec((1, _BN, _H), lambda t, nb: (t + (tin - tl), nb, 0)),
            pl.BlockSpec((1, _H), lambda t, nb: (0, 0)),
        ],
        out_specs=pl.BlockSpec((1, _BN, _H), lambda t, nb: (t, nb, 0)),
        out_shape=jax.ShapeDtypeStruct((tl, _N, _H), jnp.float32),
    )(acc, uh, res, bs.reshape(1, _H))


def _dec_body(h_ref, o_ref, x_ref, r1_ref, rb1_ref, r2_ref, rb2_ref, r3_ref,
              rb3_ref, r4_ref, rb4_ref, y_ref):
    z = h_ref[0] + o_ref[0]
    m = _lk(_dot(z, r1_ref[...]) + rb1_ref[...])
    m = _lk(_dot(m, r2_ref[...]) + rb2_ref[...])
    m = _lk(_dot(m, r3_ref[...]) + rb3_ref[...])
    m = _dot(m, r4_ref[...]) + rb4_ref[...]
    xb = x_ref[0]
    y_ref[...] = m + jnp.concatenate([xb, xb, xb, xb], axis=1)


def _decoder(h, out_acc, x3, p, tl):
    def wspec(shape):
        return pl.BlockSpec(shape, lambda nb: (0,) * len(shape))
    last = lambda nb: (tl - 1, nb, 0)
    return pl.pallas_call(
        _dec_body,
        grid=(_NB,),
        in_specs=[
            pl.BlockSpec((1, _BN, _H), last),
            pl.BlockSpec((1, _BN, _H), last),
            pl.BlockSpec((1, _BN, _F), lambda nb: (_T - 1, nb, 0)),
            wspec((_H, 2 * _H)), wspec((1, 2 * _H)),
            wspec((2 * _H, 2 * _H)), wspec((1, 2 * _H)),
            wspec((2 * _H, 2 * _H)), wspec((1, 2 * _H)),
            wspec((2 * _H, _HOR * _OUTF)), wspec((1, _HOR * _OUTF)),
        ],
        out_specs=pl.BlockSpec((_BN, _HOR * _OUTF), lambda nb: (nb, 0)),
        out_shape=jax.ShapeDtypeStruct((_N, _HOR * _OUTF), jnp.float32),
    )(h, out_acc, x3,
      p["R1"], p["rb1"].reshape(1, -1), p["R2"], p["rb2"].reshape(1, -1),
      p["R3"], p["rb3"].reshape(1, -1), p["R4"], p["rb4"].reshape(1, -1))


def kernel(x, exog, params, edge_index):
    p = params
    x3 = x[0]
    ex = exog[0, :, :, 1:2]
    wu_eff = p["cond_Wu"][0:1] / 365.0 + p["cond_Wu"][1:2]

    h = _encoder(x3, ex, p["cond_Wx"], wu_eff, p["cond_b"].reshape(1, _H),
                 p["cond_skip"], p["enc_W1"], p["enc_b1"].reshape(1, 2 * _H),
                 p["enc_W2"], p["enc_b2"].reshape(1, _H), p["node_emb"])

    pad = _EPAD - _E
    pad_dst = _N + (jnp.arange(pad, dtype=jnp.int32) % (_NPAD - _N))
    srcp = jnp.concatenate([edge_index[0], jnp.zeros((pad,), jnp.int32)])
    dstp = jnp.concatenate([edge_index[1], pad_dst])
    srcp = srcp.reshape(_NW, _NCH, _CH)
    dstp = dstp.reshape(_NW, _NCH, _CH)

    out_acc = None
    tin = _T
    tprev = _T - 1
    for l in range(2):
        lp = p["layers"][l]
        d = 2 ** (l % 2)
        tl = tin - d
        ah, bvt, uh, out_acc = _temporal(h, out_acc, lp, d, tl, tprev)
        acc = _edge_sc(tl)(ah, bvt, srcp, dstp)
        h = _combine(acc, uh, h, lp["bs"], tl, tin)
        tin = tl
        tprev = tl

    y = _decoder(h, out_acc, x3, p, tin)
    res = y.reshape(_N, _HOR, _OUTF).transpose(1, 0, 2)[None]
    return res


# speedup vs baseline: 1.7734x; 1.5816x over previous
import functools

import jax
import jax.numpy as jnp
import numpy as np
from jax import lax
from jax.experimental import pallas as pl
from jax.experimental.pallas import tpu as pltpu
from jax.experimental.pallas import tpu_sc as plsc

_N = 10000
_E = 160000
_T = 8
_F = 26
_H = 64
_HOR = 4
_OUTF = 26

_NB = 5
_BN = _N // _NB

_NC = 2
_NS = 16
_NW = _NC * _NS
_CH = 64
_PER_TILE = 5120
_EPAD = _PER_TILE * _NW
_NCH = _PER_TILE // _CH
_NPAD = 10000
_RPT = _NPAD // _NS
_NCH_LAST = (_E - (_NW - 1) * _PER_TILE) // _CH

_PLO = np.array([32 * g + k for g in range(_H // 32) for k in range(16)],
                np.int32)
_PHI = _PLO + 16


def _lk(v, s=0.01):
    return jnp.where(v >= 0, v, s * v)


def _dot(a, b):
    return jnp.dot(a, b, preferred_element_type=jnp.float32)


def _enc_body(x_ref, e_ref, wx_ref, wu_ref, b_ref, wsk_ref, w1_ref, b1_ref,
              w2_ref, b2_ref, emb_ref, o_ref):
    xb = x_ref[0]
    eb = e_ref[0]
    h = _lk(_dot(xb, wx_ref[...]) + eb * wu_ref[...] + b_ref[...])
    h = h + _dot(xb, wsk_ref[...])
    h = _lk(_dot(h, w1_ref[...]) + b1_ref[...])
    h = _lk(_dot(h, w2_ref[...]) + b2_ref[...])
    o_ref[0] = h + emb_ref[...]


def _encoder(x3, ex, wx, wu_eff, b, wsk, w1, b1, w2, b2, emb):
    def wspec(shape):
        return pl.BlockSpec(shape, lambda t, nb: (0,) * len(shape))
    return pl.pallas_call(
        _enc_body,
        grid=(_T, _NB),
        in_specs=[
            pl.BlockSpec((1, _BN, _F), lambda t, nb: (t, nb, 0)),
            pl.BlockSpec((1, _BN, 1), lambda t, nb: (t, nb, 0)),
            wspec((_F, _H)), wspec((1, _H)), wspec((1, _H)), wspec((_F, _H)),
            wspec((_H, 2 * _H)), wspec((1, 2 * _H)),
            wspec((2 * _H, _H)), wspec((1, _H)),
            pl.BlockSpec((_BN, _H), lambda t, nb: (nb, 0)),
        ],
        out_specs=pl.BlockSpec((1, _BN, _H), lambda t, nb: (t, nb, 0)),
        out_shape=jax.ShapeDtypeStruct((_T, _N, _H), jnp.float32),
    )(x3, ex, wx, wu_eff, b, wsk, w1, b1, w2, b2, emb)


def _bfpack(xlo, xhi):
    ulo = jax.lax.bitcast_convert_type(xlo, jnp.uint32)
    uhi = jax.lax.bitcast_convert_type(xhi, jnp.uint32)
    one = jnp.uint32(1)
    half = jnp.uint32(0x7FFF)
    rlo = (ulo + half + ((ulo >> 16) & one)) >> 16
    rhi = (uhi + half + ((uhi >> 16) & one)) >> 16
    return jax.lax.bitcast_convert_type(rlo | (rhi << 16), jnp.int32)


def _tmp_body_mk(has_prev):
    def body(*refs):
        if has_prev:
            (h1_ref, h0_ref, po_ref, wt1_ref, wt0_ref, bt_ref, walo_ref,
             wahi_ref, wblo_ref, wbhi_ref, wvlo_ref, wvhi_ref, wu_ref,
             ws_ref, bsk_ref, ah_ref, bv_ref, uh_ref, on_ref) = refs
        else:
            (h1_ref, h0_ref, wt1_ref, wt0_ref, bt_ref, walo_ref,
             wahi_ref, wblo_ref, wbhi_ref, wvlo_ref, wvhi_ref, wu_ref,
             ws_ref, bsk_ref, ah_ref, bv_ref, uh_ref, on_ref) = refs
        hc = _lk(_dot(h1_ref[0], wt1_ref[...]) + _dot(h0_ref[0], wt0_ref[...])
                 + bt_ref[...])
        ah_ref[...] = _bfpack(_dot(hc, walo_ref[...]), _dot(hc, wahi_ref[...]))
        bv_ref[...] = jnp.concatenate(
            [_bfpack(_dot(hc, wblo_ref[...]), _dot(hc, wbhi_ref[...])),
             _bfpack(_dot(hc, wvlo_ref[...]), _dot(hc, wvhi_ref[...]))],
            axis=1)
        uh_ref[0] = _dot(hc, wu_ref[...])
        on = _dot(hc, ws_ref[...]) + bsk_ref[...]
        if has_prev:
            on = on + po_ref[0]
        on_ref[0] = on
    return body


def _temporal(h, po, lp, d, tl, tprev):
    def wspec(shape):
        return pl.BlockSpec(shape, lambda t, nb: (0,) * len(shape))
    hspec = lambda off: pl.BlockSpec((1, _BN, _H), lambda t, nb: (t + off, nb, 0))
    tab_spec = lambda wdt: pl.BlockSpec((_BN, wdt), lambda t, nb: (t * _NB + nb, 0))
    tab_shape = lambda wdt: jax.ShapeDtypeStruct((tl * _N + 16, wdt), jnp.int32)
    seq_spec = pl.BlockSpec((1, _BN, _H), lambda t, nb: (t, nb, 0))
    seq_shape = jax.ShapeDtypeStruct((tl, _N, _H), jnp.float32)
    hw = _H // 2
    has_prev = po is not None
    po_spec = ([pl.BlockSpec((1, _BN, _H),
                             lambda t, nb: (t + (tprev - tl), nb, 0))]
               if has_prev else [])
    po_arg = [po] if has_prev else []
    return pl.pallas_call(
        _tmp_body_mk(has_prev),
        grid=(tl, _NB),
        in_specs=[
            hspec(d), hspec(0), *po_spec,
            wspec((_H, _H)), wspec((_H, _H)), wspec((1, _H)),
            wspec((_H, hw)), wspec((_H, hw)), wspec((_H, hw)),
            wspec((_H, hw)), wspec((_H, hw)), wspec((_H, hw)),
            wspec((_H, _H)), wspec((_H, _H)), wspec((1, _H)),
        ],
        out_specs=[tab_spec(hw), tab_spec(_H), seq_spec, seq_spec],
        out_shape=[tab_shape(hw), tab_shape(_H), seq_shape, seq_shape],
    )(h, h, *po_arg, lp["Wt1"], lp["Wt0"], lp["bt"].reshape(1, _H),
      lp["A"][:, _PLO], lp["A"][:, _PHI], lp["B"][:, _PLO], lp["B"][:, _PHI],
      lp["V"][:, _PLO], lp["V"][:, _PHI], lp["U"], lp["Ws"],
      lp["bskip"].reshape(1, _H))


def _edge_sc(tl):
    mesh = plsc.VectorSubcoreMesh(core_axis_name="c", subcore_axis_name="s")

    @functools.partial(
        pl.kernel,
        out_type=jax.ShapeDtypeStruct((_NC, tl, _NPAD, 128), jnp.float32),
        mesh=mesh,
        compiler_params=pltpu.CompilerParams(use_tc_tiling_on_sc=False),
        scratch_types=[
            pltpu.VMEM((_NCH, _CH), jnp.int32),
            pltpu.VMEM((_NCH, _CH), jnp.int32),
            pltpu.VMEM((_NCH, _CH), jnp.int32),
            pltpu.VMEM((_CH, _H // 2), jnp.int32),
            pltpu.VMEM((_CH, _H), jnp.int32),
            pltpu.VMEM((_CH, _H // 2), jnp.int32),
            pltpu.VMEM((_CH, _H), jnp.int32),
            pltpu.VMEM((_CH, 128), jnp.float32),
            pltpu.VMEM((_CH, 128), jnp.float32),
            pltpu.VMEM((16, 128), jnp.float32),
            pltpu.VMEM_SHARED((_NPAD, 128), jnp.float32),
            pltpu.SemaphoreType.DMA,
            pltpu.SemaphoreType.DMA,
            pltpu.SemaphoreType.DMA,
            pltpu.SemaphoreType.DMA,
        ],
    )
    def k(ah, bv, srcr, dstr, out, srco, dsti, dsto, ar0, bv0,
          ar1, bv1, ob0, ob1, zbuf, accum, sg0, sg1, ss0, ss1):
        c = lax.axis_index("c")
        s = lax.axis_index("s")
        w = c * _NS + s
        row0 = s * _RPT
        npair = jnp.where(w == _NW - 1, _NCH_LAST // 2, _NCH // 2)
        bufs = ((ar0, bv0, ob0, sg0, ss0),
                (ar1, bv1, ob1, sg1, ss1))

        pltpu.sync_copy(srcr.at[w], srco)
        pltpu.sync_copy(dstr.at[w], dsti)
        pltpu.sync_copy(dstr.at[w], dsto)

        def zrow(r, carry):
            for j in range(8):
                zbuf[r, pl.ds(16 * j, 16)] = jnp.zeros((16,), jnp.float32)
            return carry
        lax.fori_loop(0, 16, zrow, 0)

        def issue_gather(ck, b):
            arb, bvb, sgb = bufs[b][0], bufs[b][1], bufs[b][3]
            pltpu.async_copy(ah.at[dsto.at[ck]], arb, sgb)
            pltpu.async_copy(bv.at[srco.at[ck]], bvb, sgb)

        def wait_gather(b):
            arb, bvb, sgb = bufs[b][0], bufs[b][1], bufs[b][3]
            pltpu.make_async_copy(ah.at[dsto.at[0]], arb, sgb).wait()
            pltpu.make_async_copy(bv.at[srco.at[0]], bvb, sgb).wait()

        def wait_scatter(b):
            obb, ssb = bufs[b][2], bufs[b][4]
            pltpu.make_async_copy(obb, accum.at[dsti.at[0]], ssb).wait()

        def tt_body(tt, carry):
            def zcp(kk, cc):
                pltpu.sync_copy(zbuf, accum.at[pl.ds(row0 + kk * 16, 16)])
                return cc
            lax.fori_loop(0, _RPT // 16, zcp, 0)
            rem = _RPT % 16
            if rem:
                pltpu.sync_copy(zbuf.at[pl.ds(0, rem)],
                                accum.at[pl.ds(row0 + _RPT - rem, rem)])
            plsc.subcore_barrier()

            def off(ck2, c2):
                for j in range(_CH // 16):
                    sl = pl.ds(16 * j, 16)
                    srco[ck2, sl] = srco[ck2, sl] + _N
                    dsto[ck2, sl] = dsto[ck2, sl] + _N
                return c2

            @pl.when(tt > 0)
            def _():
                lax.fori_loop(0, _NCH, off, 0)

            issue_gather(0, 0)

            def pair(i, c2):
                for b in (0, 1):
                    ck = 2 * i + b
                    if b == 0:
                        issue_gather(ck + 1, 1)
                    else:
                        @pl.when(i < npair - 1)
                        def _():
                            issue_gather(ck + 1, 0)
                    wait_gather(b)

                    @pl.when(i > 0)
                    def _():
                        wait_scatter(b)

                    arb, bvb, obb, _, ssb = bufs[b]

                    def _unp(word):
                        lo = jax.lax.bitcast_convert_type(word << 16,
                                                          jnp.float32)
                        hi = jax.lax.bitcast_convert_type(word, jnp.float32)
                        return lo, hi

                    @plsc.parallel_loop(0, _CH, 1, unroll=4)
                    def ebody(e2):
                        for j in range(_H // 32):
                            sl = pl.ds(16 * j, 16)
                            a0, a1 = _unp(arb[e2, sl])
                            b0, b1 = _unp(bvb[e2, sl])
                            v0, v1 = _unp(bvb[e2, pl.ds(_H // 2 + 16 * j, 16)])
                            g0 = 1.0 / (1.0 + jnp.exp(-(a0 + b0)))
                            g1 = 1.0 / (1.0 + jnp.exp(-(a1 + b1)))
                            obb[e2, pl.ds(32 * j, 16)] = g0 * v0
                            obb[e2, pl.ds(32 * j + 16, 16)] = g1 * v1
                            obb[e2, pl.ds(_H + 32 * j, 16)] = g0
                            obb[e2, pl.ds(_H + 32 * j + 16, 16)] = g1

                    pltpu.async_copy(obb, accum.at[dsti.at[ck]], ssb,
                                     add=True)
                return c2
            lax.fori_loop(0, npair, pair, 0)
            wait_scatter(0)
            wait_scatter(1)
            plsc.subcore_barrier()

            pltpu.sync_copy(accum.at[pl.ds(row0, _RPT)],
                            out.at[c, tt, pl.ds(row0, _RPT)])
            plsc.subcore_barrier()
            return carry
        lax.fori_loop(0, tl, tt_body, 0)

    return k


def _cmb_body(acc_ref, uh_ref, res_ref, bs_ref, o_ref):
    sm = (acc_ref[0, 0].astype(jnp.float32)
          + acc_ref[1, 0].astype(jnp.float32))
    num = sm[:, :_H]
    den = sm[:, _H:]
    g = uh_ref[0] + num / (den + 1e-6) + bs_ref[...]
    o_ref[0] = jnp.where(g >= 0, g, 0.1 * g) + res_ref[0]


def _combine(acc, uh, res, bs, tl, tin):
    return pl.pallas_call(
        _cmb_body,
        grid=(tl, _NB),
        in_specs=[
            pl.BlockSpec((_NC, 1, _BN, 128), lambda t, nb: (0, t, nb, 0)),
            pl.BlockSpec((1, _BN, _H), lambda t, nb: (t, nb, 0)),
            pl.BlockSpec((1, _BN, _H), lambda t, nb: (t + (tin - tl), nb, 0)),
            pl.BlockSpec((1, _H), lambda t, nb: (0, 0)),
        ],
        out_specs=pl.BlockSpec((1, _BN, _H), lambda t, nb: (t, nb, 0)),
        out_shape=jax.ShapeDtypeStruct((tl, _N, _H), jnp.float32),
    )(acc, uh, res, bs.reshape(1, _H))


def _dec_body(h_ref, o_ref, x_ref, r1_ref, rb1_ref, r2_ref, rb2_ref, r3_ref,
              rb3_ref, r4_ref, rb4_ref, y_ref):
    z = h_ref[0] + o_ref[0]
    m = _lk(_dot(z, r1_ref[...]) + rb1_ref[...])
    m = _lk(_dot(m, r2_ref[...]) + rb2_ref[...])
    m = _lk(_dot(m, r3_ref[...]) + rb3_ref[...])
    m = _dot(m, r4_ref[...]) + rb4_ref[...]
    xb = x_ref[0]
    y_ref[...] = m + jnp.concatenate([xb, xb, xb, xb], axis=1)


def _decoder(h, out_acc, x3, p, tl):
    def wspec(shape):
        return pl.BlockSpec(shape, lambda nb: (0,) * len(shape))
    last = lambda nb: (tl - 1, nb, 0)
    return pl.pallas_call(
        _dec_body,
        grid=(_NB,),
        in_specs=[
            pl.BlockSpec((1, _BN, _H), last),
            pl.BlockSpec((1, _BN, _H), last),
            pl.BlockSpec((1, _BN, _F), lambda nb: (_T - 1, nb, 0)),
            wspec((_H, 2 * _H)), wspec((1, 2 * _H)),
            wspec((2 * _H, 2 * _H)), wspec((1, 2 * _H)),
            wspec((2 * _H, 2 * _H)), wspec((1, 2 * _H)),
            wspec((2 * _H, _HOR * _OUTF)), wspec((1, _HOR * _OUTF)),
        ],
        out_specs=pl.BlockSpec((_BN, _HOR * _OUTF), lambda nb: (nb, 0)),
        out_shape=jax.ShapeDtypeStruct((_N, _HOR * _OUTF), jnp.float32),
    )(h, out_acc, x3,
      p["R1"], p["rb1"].reshape(1, -1), p["R2"], p["rb2"].reshape(1, -1),
      p["R3"], p["rb3"].reshape(1, -1), p["R4"], p["rb4"].reshape(1, -1))


def kernel(x, exog, params, edge_index):
    p = params
    x3 = x[0]
    ex = exog[0, :, :, 1:2]
    wu_eff = p["cond_Wu"][0:1] / 365.0 + p["cond_Wu"][1:2]

    h = _encoder(x3, ex, p["cond_Wx"], wu_eff, p["cond_b"].reshape(1, _H),
                 p["cond_skip"], p["enc_W1"], p["enc_b1"].reshape(1, 2 * _H),
                 p["enc_W2"], p["enc_b2"].reshape(1, _H), p["node_emb"])

    pad = _EPAD - _E
    srcp = jnp.concatenate([edge_index[0], jnp.zeros((pad,), jnp.int32)])
    dstp = jnp.concatenate([edge_index[1], jnp.zeros((pad,), jnp.int32)])
    srcp = srcp.reshape(_NW, _NCH, _CH)
    dstp = dstp.reshape(_NW, _NCH, _CH)

    out_acc = None
    tin = _T
    tprev = _T - 1
    for l in range(2):
        lp = p["layers"][l]
        d = 2 ** (l % 2)
        tl = tin - d
        ah, bvt, uh, out_acc = _temporal(h, out_acc, lp, d, tl, tprev)
        acc = _edge_sc(tl)(ah, bvt, srcp, dstp)
        h = _combine(acc, uh, h, lp["bs"], tl, tin)
        tin = tl
        tprev = tl

    y = _decoder(h, out_acc, x3, p, tin)
    res = y.reshape(_N, _HOR, _OUTF).transpose(1, 0, 2)[None]
    return res


# speedup vs baseline: 1.8048x; 1.0177x over previous
import functools

import jax
import jax.numpy as jnp
import numpy as np
from jax import lax
from jax.experimental import pallas as pl
from jax.experimental.pallas import tpu as pltpu
from jax.experimental.pallas import tpu_sc as plsc

_N = 10000
_E = 160000
_T = 8
_F = 26
_H = 64
_HOR = 4
_OUTF = 26

_NB = 5
_BN = _N // _NB

_NC = 2
_NS = 16
_NW = _NC * _NS
_CH = 64
_PER_TILE = 5120
_EPAD = _PER_TILE * _NW
_NCH = _PER_TILE // _CH
_NPAD = 10000
_RPT = _NPAD // _NS
_NCH_LAST = (_E - (_NW - 1) * _PER_TILE) // _CH

_PLO = np.array([32 * g + k for g in range(_H // 32) for k in range(16)],
                np.int32)
_PHI = _PLO + 16


def _lk(v, s=0.01):
    return jnp.where(v >= 0, v, s * v)


def _dot(a, b):
    return jnp.dot(a, b, preferred_element_type=jnp.float32)


def _enc_body(x_ref, e_ref, wx_ref, wu_ref, b_ref, wsk_ref, w1_ref, b1_ref,
              w2_ref, b2_ref, emb_ref, o_ref):
    xb = x_ref[0]
    eb = e_ref[0]
    h = _lk(_dot(xb, wx_ref[...]) + eb * wu_ref[...] + b_ref[...])
    h = h + _dot(xb, wsk_ref[...])
    h = _lk(_dot(h, w1_ref[...]) + b1_ref[...])
    h = _lk(_dot(h, w2_ref[...]) + b2_ref[...])
    o_ref[0] = h + emb_ref[...]


def _encoder(x3, ex, wx, wu_eff, b, wsk, w1, b1, w2, b2, emb):
    def wspec(shape):
        return pl.BlockSpec(shape, lambda t, nb: (0,) * len(shape))
    return pl.pallas_call(
        _enc_body,
        grid=(_T, _NB),
        in_specs=[
            pl.BlockSpec((1, _BN, _F), lambda t, nb: (t, nb, 0)),
            pl.BlockSpec((1, _BN, 1), lambda t, nb: (t, nb, 0)),
            wspec((_F, _H)), wspec((1, _H)), wspec((1, _H)), wspec((_F, _H)),
            wspec((_H, 2 * _H)), wspec((1, 2 * _H)),
            wspec((2 * _H, _H)), wspec((1, _H)),
            pl.BlockSpec((_BN, _H), lambda t, nb: (nb, 0)),
        ],
        out_specs=pl.BlockSpec((1, _BN, _H), lambda t, nb: (t, nb, 0)),
        out_shape=jax.ShapeDtypeStruct((_T, _N, _H), jnp.float32),
    )(x3, ex, wx, wu_eff, b, wsk, w1, b1, w2, b2, emb)


def _bfpack(xlo, xhi):
    ulo = jax.lax.bitcast_convert_type(xlo, jnp.uint32)
    uhi = jax.lax.bitcast_convert_type(xhi, jnp.uint32)
    one = jnp.uint32(1)
    half = jnp.uint32(0x7FFF)
    rlo = (ulo + half + ((ulo >> 16) & one)) >> 16
    rhi = (uhi + half + ((uhi >> 16) & one)) >> 16
    return jax.lax.bitcast_convert_type(rlo | (rhi << 16), jnp.int32)


def _tmp_body_mk(has_prev):
    def body(*refs):
        if has_prev:
            (h1_ref, h0_ref, po_ref, wt1_ref, wt0_ref, bt_ref, walo_ref,
             wahi_ref, wblo_ref, wbhi_ref, wvlo_ref, wvhi_ref, wu_ref,
             ws_ref, bsk_ref, ah_ref, bv_ref, uh_ref, on_ref) = refs
        else:
            (h1_ref, h0_ref, wt1_ref, wt0_ref, bt_ref, walo_ref,
             wahi_ref, wblo_ref, wbhi_ref, wvlo_ref, wvhi_ref, wu_ref,
             ws_ref, bsk_ref, ah_ref, bv_ref, uh_ref, on_ref) = refs
        hc = _lk(_dot(h1_ref[0], wt1_ref[...]) + _dot(h0_ref[0], wt0_ref[...])
                 + bt_ref[...])
        ah_ref[...] = _bfpack(_dot(hc, walo_ref[...]), _dot(hc, wahi_ref[...]))
        bv_ref[...] = jnp.concatenate(
            [_bfpack(_dot(hc, wblo_ref[...]), _dot(hc, wbhi_ref[...])),
             _bfpack(_dot(hc, wvlo_ref[...]), _dot(hc, wvhi_ref[...]))],
            axis=1)
        uh_ref[0] = _dot(hc, wu_ref[...])
        on = _dot(hc, ws_ref[...]) + bsk_ref[...]
        if has_prev:
            on = on + po_ref[0]
        on_ref[0] = on
    return body


def _temporal(h, po, lp, d, tl, tprev):
    def wspec(shape):
        return pl.BlockSpec(shape, lambda t, nb: (0,) * len(shape))
    hspec = lambda off: pl.BlockSpec((1, _BN, _H), lambda t, nb: (t + off, nb, 0))
    tab_spec = lambda wdt: pl.BlockSpec((_BN, wdt), lambda t, nb: (t * _NB + nb, 0))
    tab_shape = lambda wdt: jax.ShapeDtypeStruct((tl * _N + 16, wdt), jnp.int32)
    seq_spec = pl.BlockSpec((1, _BN, _H), lambda t, nb: (t, nb, 0))
    seq_shape = jax.ShapeDtypeStruct((tl, _N, _H), jnp.float32)
    hw = _H // 2
    has_prev = po is not None
    po_spec = ([pl.BlockSpec((1, _BN, _H),
                             lambda t, nb: (t + (tprev - tl), nb, 0))]
               if has_prev else [])
    po_arg = [po] if has_prev else []
    return pl.pallas_call(
        _tmp_body_mk(has_prev),
        grid=(tl, _NB),
        in_specs=[
            hspec(d), hspec(0), *po_spec,
            wspec((_H, _H)), wspec((_H, _H)), wspec((1, _H)),
            wspec((_H, hw)), wspec((_H, hw)), wspec((_H, hw)),
            wspec((_H, hw)), wspec((_H, hw)), wspec((_H, hw)),
            wspec((_H, _H)), wspec((_H, _H)), wspec((1, _H)),
        ],
        out_specs=[tab_spec(hw), tab_spec(_H), seq_spec, seq_spec],
        out_shape=[tab_shape(hw), tab_shape(_H), seq_shape, seq_shape],
    )(h, h, *po_arg, lp["Wt1"], lp["Wt0"], lp["bt"].reshape(1, _H),
      lp["A"][:, _PLO], lp["A"][:, _PHI], lp["B"][:, _PLO], lp["B"][:, _PHI],
      lp["V"][:, _PLO], lp["V"][:, _PHI], lp["U"], lp["Ws"],
      lp["bskip"].reshape(1, _H))


def _edge_sc(tl):
    mesh = plsc.VectorSubcoreMesh(core_axis_name="c", subcore_axis_name="s")

    @functools.partial(
        pl.kernel,
        out_type=jax.ShapeDtypeStruct((_NC, tl, _NPAD, 128), jnp.float32),
        mesh=mesh,
        compiler_params=pltpu.CompilerParams(use_tc_tiling_on_sc=False),
        scratch_types=[
            pltpu.VMEM((_NCH, _CH), jnp.int32),
            pltpu.VMEM((_NCH, _CH), jnp.int32),
            pltpu.VMEM((_NCH, _CH), jnp.int32),
            pltpu.VMEM((_CH, _H // 2), jnp.int32),
            pltpu.VMEM((_CH, _H), jnp.int32),
            pltpu.VMEM((_CH, _H // 2), jnp.int32),
            pltpu.VMEM((_CH, _H), jnp.int32),
            pltpu.VMEM((_CH, 128), jnp.float32),
            pltpu.VMEM((_CH, 128), jnp.float32),
            pltpu.VMEM((32, 128), jnp.float32),
            pltpu.VMEM_SHARED((_NPAD, 128), jnp.float32),
            pltpu.SemaphoreType.DMA,
            pltpu.SemaphoreType.DMA,
            pltpu.SemaphoreType.DMA,
            pltpu.SemaphoreType.DMA,
            pltpu.SemaphoreType.DMA,
        ],
    )
    def k(ah, bv, srcr, dstr, out, srco, dsti, dsto, ar0, bv0,
          ar1, bv1, ob0, ob1, zbuf, accum, sg0, sg1, ss0, ss1, sz):
        c = lax.axis_index("c")
        s = lax.axis_index("s")
        w = c * _NS + s
        row0 = s * _RPT
        npair = jnp.where(w == _NW - 1, _NCH_LAST // 2, _NCH // 2)
        bufs = ((ar0, bv0, ob0, sg0, ss0),
                (ar1, bv1, ob1, sg1, ss1))

        pltpu.sync_copy(srcr.at[w], srco)
        pltpu.sync_copy(dstr.at[w], dsti)
        pltpu.sync_copy(dstr.at[w], dsto)

        def zrow(r, carry):
            for j in range(8):
                zbuf[r, pl.ds(16 * j, 16)] = jnp.zeros((16,), jnp.float32)
            return carry
        lax.fori_loop(0, 32, zrow, 0)

        def issue_gather(ck, b):
            arb, bvb, sgb = bufs[b][0], bufs[b][1], bufs[b][3]
            pltpu.async_copy(ah.at[dsto.at[ck]], arb, sgb)
            pltpu.async_copy(bv.at[srco.at[ck]], bvb, sgb)

        def wait_gather(b):
            arb, bvb, sgb = bufs[b][0], bufs[b][1], bufs[b][3]
            pltpu.make_async_copy(ah.at[dsto.at[0]], arb, sgb).wait()
            pltpu.make_async_copy(bv.at[srco.at[0]], bvb, sgb).wait()

        def wait_scatter(b):
            obb, ssb = bufs[b][2], bufs[b][4]
            pltpu.make_async_copy(obb, accum.at[dsti.at[0]], ssb).wait()

        def tt_body(tt, carry):
            zrem = _RPT % 32

            def zcp(kk, cc):
                pltpu.async_copy(zbuf, accum.at[pl.ds(row0 + kk * 32, 32)],
                                 sz)
                return cc
            lax.fori_loop(0, _RPT // 32, zcp, 0)
            if zrem:
                pltpu.async_copy(zbuf.at[pl.ds(0, zrem)],
                                 accum.at[pl.ds(row0 + _RPT - zrem, zrem)],
                                 sz)

            def zwt(kk, cc):
                pltpu.make_async_copy(zbuf,
                                      accum.at[pl.ds(row0, 32)], sz).wait()
                return cc
            lax.fori_loop(0, _RPT // 32, zwt, 0)
            if zrem:
                pltpu.make_async_copy(zbuf.at[pl.ds(0, zrem)],
                                      accum.at[pl.ds(row0, zrem)], sz).wait()
            plsc.subcore_barrier()

            def off(ck2, c2):
                for j in range(_CH // 16):
                    sl = pl.ds(16 * j, 16)
                    srco[ck2, sl] = srco[ck2, sl] + _N
                    dsto[ck2, sl] = dsto[ck2, sl] + _N
                return c2

            @pl.when(tt > 0)
            def _():
                lax.fori_loop(0, _NCH, off, 0)

            issue_gather(0, 0)

            def pair(i, c2):
                for b in (0, 1):
                    ck = 2 * i + b
                    if b == 0:
                        issue_gather(ck + 1, 1)
                    else:
                        @pl.when(i < npair - 1)
                        def _():
                            issue_gather(ck + 1, 0)
                    wait_gather(b)

                    @pl.when(i > 0)
                    def _():
                        wait_scatter(b)

                    arb, bvb, obb, _, ssb = bufs[b]

                    def _unp(word):
                        lo = jax.lax.bitcast_convert_type(word << 16,
                                                          jnp.float32)
                        hi = jax.lax.bitcast_convert_type(word, jnp.float32)
                        return lo, hi

                    @plsc.parallel_loop(0, _CH, 1, unroll=4)
                    def ebody(e2):
                        for j in range(_H // 32):
                            sl = pl.ds(16 * j, 16)
                            a0, a1 = _unp(arb[e2, sl])
                            b0, b1 = _unp(bvb[e2, sl])
                            v0, v1 = _unp(bvb[e2, pl.ds(_H // 2 + 16 * j, 16)])
                            g0 = 1.0 / (1.0 + jnp.exp(-(a0 + b0)))
                            g1 = 1.0 / (1.0 + jnp.exp(-(a1 + b1)))
                            obb[e2, pl.ds(32 * j, 16)] = g0 * v0
                            obb[e2, pl.ds(32 * j + 16, 16)] = g1 * v1
                            obb[e2, pl.ds(_H + 32 * j, 16)] = g0
                            obb[e2, pl.ds(_H + 32 * j + 16, 16)] = g1

                    pltpu.async_copy(obb, accum.at[dsti.at[ck]], ssb,
                                     add=True)
                return c2
            lax.fori_loop(0, npair, pair, 0)
            wait_scatter(0)
            wait_scatter(1)
            plsc.subcore_barrier()

            pltpu.sync_copy(accum.at[pl.ds(row0, _RPT)],
                            out.at[c, tt, pl.ds(row0, _RPT)])
            return carry
        lax.fori_loop(0, tl, tt_body, 0)

    return k


def _cmb_body(acc_ref, uh_ref, res_ref, bs_ref, o_ref):
    sm = (acc_ref[0, 0].astype(jnp.float32)
          + acc_ref[1, 0].astype(jnp.float32))
    num = sm[:, :_H]
    den = sm[:, _H:]
    g = uh_ref[0] + num / (den + 1e-6) + bs_ref[...]
    o_ref[0] = jnp.where(g >= 0, g, 0.1 * g) + res_ref[0]


def _combine(acc, uh, res, bs, tl, tin):
    return pl.pallas_call(
        _cmb_body,
        grid=(tl, _NB),
        in_specs=[
            pl.BlockSpec((_NC, 1, _BN, 128), lambda t, nb: (0, t, nb, 0)),
            pl.BlockSpec((1, _BN, _H), lambda t, nb: (t, nb, 0)),
            pl.BlockSpec((1, _BN, _H), lambda t, nb: (t + (tin - tl), nb, 0)),
            pl.BlockSpec((1, _H), lambda t, nb: (0, 0)),
        ],
        out_specs=pl.BlockSpec((1, _BN, _H), lambda t, nb: (t, nb, 0)),
        out_shape=jax.ShapeDtypeStruct((tl, _N, _H), jnp.float32),
    )(acc, uh, res, bs.reshape(1, _H))


def _dec_body(h_ref, o_ref, x_ref, r1_ref, rb1_ref, r2_ref, rb2_ref, r3_ref,
              rb3_ref, r4_ref, rb4_ref, y_ref):
    z = h_ref[0] + o_ref[0]
    m = _lk(_dot(z, r1_ref[...]) + rb1_ref[...])
    m = _lk(_dot(m, r2_ref[...]) + rb2_ref[...])
    m = _lk(_dot(m, r3_ref[...]) + rb3_ref[...])
    m = _dot(m, r4_ref[...]) + rb4_ref[...]
    xb = x_ref[0]
    y_ref[...] = m + jnp.concatenate([xb, xb, xb, xb], axis=1)


def _decoder(h, out_acc, x3, p, tl):
    def wspec(shape):
        return pl.BlockSpec(shape, lambda nb: (0,) * len(shape))
    last = lambda nb: (tl - 1, nb, 0)
    return pl.pallas_call(
        _dec_body,
        grid=(_NB,),
        in_specs=[
            pl.BlockSpec((1, _BN, _H), last),
            pl.BlockSpec((1, _BN, _H), last),
            pl.BlockSpec((1, _BN, _F), lambda nb: (_T - 1, nb, 0)),
            wspec((_H, 2 * _H)), wspec((1, 2 * _H)),
            wspec((2 * _H, 2 * _H)), wspec((1, 2 * _H)),
            wspec((2 * _H, 2 * _H)), wspec((1, 2 * _H)),
            wspec((2 * _H, _HOR * _OUTF)), wspec((1, _HOR * _OUTF)),
        ],
        out_specs=pl.BlockSpec((_BN, _HOR * _OUTF), lambda nb: (nb, 0)),
        out_shape=jax.ShapeDtypeStruct((_N, _HOR * _OUTF), jnp.float32),
    )(h, out_acc, x3,
      p["R1"], p["rb1"].reshape(1, -1), p["R2"], p["rb2"].reshape(1, -1),
      p["R3"], p["rb3"].reshape(1, -1), p["R4"], p["rb4"].reshape(1, -1))


def kernel(x, exog, params, edge_index):
    p = params
    x3 = x[0]
    ex = exog[0, :, :, 1:2]
    wu_eff = p["cond_Wu"][0:1] / 365.0 + p["cond_Wu"][1:2]

    h = _encoder(x3, ex, p["cond_Wx"], wu_eff, p["cond_b"].reshape(1, _H),
                 p["cond_skip"], p["enc_W1"], p["enc_b1"].reshape(1, 2 * _H),
                 p["enc_W2"], p["enc_b2"].reshape(1, _H), p["node_emb"])

    pad = _EPAD - _E
    srcp = jnp.concatenate([edge_index[0], jnp.zeros((pad,), jnp.int32)])
    dstp = jnp.concatenate([edge_index[1], jnp.zeros((pad,), jnp.int32)])
    srcp = srcp.reshape(_NW, _NCH, _CH)
    dstp = dstp.reshape(_NW, _NCH, _CH)

    out_acc = None
    tin = _T
    tprev = _T - 1
    for l in range(2):
        lp = p["layers"][l]
        d = 2 ** (l % 2)
        tl = tin - d
        ah, bvt, uh, out_acc = _temporal(h, out_acc, lp, d, tl, tprev)
        acc = _edge_sc(tl)(ah, bvt, srcp, dstp)
        h = _combine(acc, uh, h, lp["bs"], tl, tin)
        tin = tl
        tprev = tl

    y = _decoder(h, out_acc, x3, p, tin)
    res = y.reshape(_N, _HOR, _OUTF).transpose(1, 0, 2)[None]
    return res


# speedup vs baseline: 1.8594x; 1.0303x over previous
import functools

import jax
import jax.numpy as jnp
import numpy as np
from jax import lax
from jax.experimental import pallas as pl
from jax.experimental.pallas import tpu as pltpu
from jax.experimental.pallas import tpu_sc as plsc

_N = 10000
_E = 160000
_T = 8
_F = 26
_H = 64
_HOR = 4
_OUTF = 26

_NB = 5
_BN = _N // _NB

_NC = 2
_NS = 16
_NW = _NC * _NS
_CH = 64
_PER_TILE = 5120
_EPAD = _PER_TILE * _NW
_NCH = _PER_TILE // _CH
_NPAD = 10000
_RPT = _NPAD // _NS
_NCH_LAST = (_E - (_NW - 1) * _PER_TILE) // _CH

_PLO = np.array([32 * g + k for g in range(_H // 32) for k in range(16)],
                np.int32)
_PHI = _PLO + 16


def _lk(v, s=0.01):
    return jnp.where(v >= 0, v, s * v)


def _dot(a, b):
    return jnp.dot(a, b, preferred_element_type=jnp.float32)


def _enc_body(x_ref, e_ref, wx_ref, wu_ref, b_ref, wsk_ref, w1_ref, b1_ref,
              w2_ref, b2_ref, emb_ref, o_ref):
    xb = x_ref[0]
    eb = e_ref[0]
    h = _lk(_dot(xb, wx_ref[...]) + eb * wu_ref[...] + b_ref[...])
    h = h + _dot(xb, wsk_ref[...])
    h = _lk(_dot(h, w1_ref[...]) + b1_ref[...])
    h = _lk(_dot(h, w2_ref[...]) + b2_ref[...])
    o_ref[0] = h + emb_ref[...]


def _encoder(x3, ex, wx, wu_eff, b, wsk, w1, b1, w2, b2, emb):
    def wspec(shape):
        return pl.BlockSpec(shape, lambda t, nb: (0,) * len(shape))
    return pl.pallas_call(
        _enc_body,
        grid=(_T, _NB),
        in_specs=[
            pl.BlockSpec((1, _BN, _F), lambda t, nb: (t, nb, 0)),
            pl.BlockSpec((1, _BN, 1), lambda t, nb: (t, nb, 0)),
            wspec((_F, _H)), wspec((1, _H)), wspec((1, _H)), wspec((_F, _H)),
            wspec((_H, 2 * _H)), wspec((1, 2 * _H)),
            wspec((2 * _H, _H)), wspec((1, _H)),
            pl.BlockSpec((_BN, _H), lambda t, nb: (nb, 0)),
        ],
        out_specs=pl.BlockSpec((1, _BN, _H), lambda t, nb: (t, nb, 0)),
        out_shape=jax.ShapeDtypeStruct((_T, _N, _H), jnp.float32),
    )(x3, ex, wx, wu_eff, b, wsk, w1, b1, w2, b2, emb)


def _bfpack(xlo, xhi):
    ulo = jax.lax.bitcast_convert_type(xlo, jnp.uint32)
    uhi = jax.lax.bitcast_convert_type(xhi, jnp.uint32)
    one = jnp.uint32(1)
    half = jnp.uint32(0x7FFF)
    rlo = (ulo + half + ((ulo >> 16) & one)) >> 16
    rhi = (uhi + half + ((uhi >> 16) & one)) >> 16
    return jax.lax.bitcast_convert_type(rlo | (rhi << 16), jnp.int32)


def _tmp_body_mk(has_prev):
    def body(*refs):
        if has_prev:
            (h1_ref, h0_ref, po_ref, wt1_ref, wt0_ref, bt_ref, walo_ref,
             wahi_ref, wblo_ref, wbhi_ref, wvlo_ref, wvhi_ref, wu_ref,
             ws_ref, bsk_ref, ah_ref, bv_ref, uh_ref, on_ref) = refs
        else:
            (h1_ref, h0_ref, wt1_ref, wt0_ref, bt_ref, walo_ref,
             wahi_ref, wblo_ref, wbhi_ref, wvlo_ref, wvhi_ref, wu_ref,
             ws_ref, bsk_ref, ah_ref, bv_ref, uh_ref, on_ref) = refs
        hc = _lk(_dot(h1_ref[0], wt1_ref[...]) + _dot(h0_ref[0], wt0_ref[...])
                 + bt_ref[...])
        ah_ref[...] = _bfpack(_dot(hc, walo_ref[...]), _dot(hc, wahi_ref[...]))
        bv_ref[...] = jnp.concatenate(
            [_bfpack(_dot(hc, wblo_ref[...]), _dot(hc, wbhi_ref[...])),
             _bfpack(_dot(hc, wvlo_ref[...]), _dot(hc, wvhi_ref[...]))],
            axis=1)
        uh_ref[0] = _dot(hc, wu_ref[...])
        on = _dot(hc, ws_ref[...]) + bsk_ref[...]
        if has_prev:
            on = on + po_ref[0]
        on_ref[0] = on
    return body


def _temporal(h, po, lp, d, tl, tprev):
    def wspec(shape):
        return pl.BlockSpec(shape, lambda t, nb: (0,) * len(shape))
    hspec = lambda off: pl.BlockSpec((1, _BN, _H), lambda t, nb: (t + off, nb, 0))
    tab_spec = lambda wdt: pl.BlockSpec((_BN, wdt), lambda t, nb: (t * _NB + nb, 0))
    tab_shape = lambda wdt: jax.ShapeDtypeStruct((tl * _N + 16, wdt), jnp.int32)
    seq_spec = pl.BlockSpec((1, _BN, _H), lambda t, nb: (t, nb, 0))
    seq_shape = jax.ShapeDtypeStruct((tl, _N, _H), jnp.float32)
    hw = _H // 2
    has_prev = po is not None
    po_spec = ([pl.BlockSpec((1, _BN, _H),
                             lambda t, nb: (t + (tprev - tl), nb, 0))]
               if has_prev else [])
    po_arg = [po] if has_prev else []
    return pl.pallas_call(
        _tmp_body_mk(has_prev),
        grid=(tl, _NB),
        in_specs=[
            hspec(d), hspec(0), *po_spec,
            wspec((_H, _H)), wspec((_H, _H)), wspec((1, _H)),
            wspec((_H, hw)), wspec((_H, hw)), wspec((_H, hw)),
            wspec((_H, hw)), wspec((_H, hw)), wspec((_H, hw)),
            wspec((_H, _H)), wspec((_H, _H)), wspec((1, _H)),
        ],
        out_specs=[tab_spec(hw), tab_spec(_H), seq_spec, seq_spec],
        out_shape=[tab_shape(hw), tab_shape(_H), seq_shape, seq_shape],
    )(h, h, *po_arg, lp["Wt1"], lp["Wt0"], lp["bt"].reshape(1, _H),
      lp["A"][:, _PLO], lp["A"][:, _PHI], lp["B"][:, _PLO], lp["B"][:, _PHI],
      lp["V"][:, _PLO], lp["V"][:, _PHI], lp["U"], lp["Ws"],
      lp["bskip"].reshape(1, _H))


def _edge_sc(tl):
    mesh = plsc.VectorSubcoreMesh(core_axis_name="c", subcore_axis_name="s")

    @functools.partial(
        pl.kernel,
        out_type=jax.ShapeDtypeStruct((_NC, tl, _NPAD, 128), jnp.float32),
        mesh=mesh,
        compiler_params=pltpu.CompilerParams(use_tc_tiling_on_sc=False),
        scratch_types=[
            pltpu.VMEM((_NCH, _CH), jnp.int32),
            pltpu.VMEM((_NCH, _CH), jnp.int32),
            pltpu.VMEM((_NCH, _CH), jnp.int32),
            pltpu.VMEM((_CH, _H // 2), jnp.int32),
            pltpu.VMEM((_CH, _H), jnp.int32),
            pltpu.VMEM((_CH, _H // 2), jnp.int32),
            pltpu.VMEM((_CH, _H), jnp.int32),
            pltpu.VMEM((_CH, 128), jnp.float32),
            pltpu.VMEM((_CH, 128), jnp.float32),
            pltpu.VMEM((32, 128), jnp.float32),
            pltpu.VMEM_SHARED((_NPAD, 128), jnp.float32),
            pltpu.SemaphoreType.DMA,
            pltpu.SemaphoreType.DMA,
            pltpu.SemaphoreType.DMA,
            pltpu.SemaphoreType.DMA,
            pltpu.SemaphoreType.DMA,
        ],
    )
    def k(ah, bv, srcr, dstr, out, srco, dsti, dsto, ar0, bv0,
          ar1, bv1, ob0, ob1, zbuf, accum, sg0, sg1, ss0, ss1, sz):
        c = lax.axis_index("c")
        s = lax.axis_index("s")
        w = c * _NS + s
        row0 = s * _RPT
        npair = jnp.where(w == _NW - 1, _NCH_LAST // 2, _NCH // 2)
        bufs = ((ar0, bv0, ob0, sg0, ss0),
                (ar1, bv1, ob1, sg1, ss1))

        pltpu.sync_copy(srcr.at[w], srco)
        pltpu.sync_copy(dstr.at[w], dsti)
        pltpu.sync_copy(dstr.at[w], dsto)

        def zrow(r, carry):
            for j in range(8):
                zbuf[r, pl.ds(16 * j, 16)] = jnp.zeros((16,), jnp.float32)
            return carry
        lax.fori_loop(0, 32, zrow, 0)

        def issue_gather(ck, b):
            arb, bvb, sgb = bufs[b][0], bufs[b][1], bufs[b][3]
            pltpu.async_copy(ah.at[dsto.at[ck]], arb, sgb)
            pltpu.async_copy(bv.at[srco.at[ck]], bvb, sgb)

        def wait_gather(b):
            arb, bvb, sgb = bufs[b][0], bufs[b][1], bufs[b][3]
            pltpu.make_async_copy(ah.at[dsto.at[0]], arb, sgb).wait()
            pltpu.make_async_copy(bv.at[srco.at[0]], bvb, sgb).wait()

        def wait_scatter(b):
            obb, ssb = bufs[b][2], bufs[b][4]
            pltpu.make_async_copy(obb, accum.at[dsti.at[0]], ssb).wait()

        def tt_body(tt, carry):
            zrem = _RPT % 32

            def zcp(kk, cc):
                pltpu.async_copy(zbuf, accum.at[pl.ds(row0 + kk * 32, 32)],
                                 sz)
                return cc
            lax.fori_loop(0, _RPT // 32, zcp, 0)
            if zrem:
                pltpu.async_copy(zbuf.at[pl.ds(0, zrem)],
                                 accum.at[pl.ds(row0 + _RPT - zrem, zrem)],
                                 sz)

            def zwt(kk, cc):
                pltpu.make_async_copy(zbuf,
                                      accum.at[pl.ds(row0, 32)], sz).wait()
                return cc
            lax.fori_loop(0, _RPT // 32, zwt, 0)
            if zrem:
                pltpu.make_async_copy(zbuf.at[pl.ds(0, zrem)],
                                      accum.at[pl.ds(row0, zrem)], sz).wait()
            plsc.subcore_barrier()

            def off(ck2, c2):
                for j in range(_CH // 16):
                    sl = pl.ds(16 * j, 16)
                    srco[ck2, sl] = srco[ck2, sl] + _N
                    dsto[ck2, sl] = dsto[ck2, sl] + _N
                return c2

            @pl.when(tt > 0)
            def _():
                lax.fori_loop(0, _NCH, off, 0)

            issue_gather(0, 0)

            def pair(i, c2):
                for b in (0, 1):
                    ck = 2 * i + b
                    if b == 0:
                        issue_gather(ck + 1, 1)
                    else:
                        @pl.when(i < npair - 1)
                        def _():
                            issue_gather(ck + 1, 0)
                    wait_gather(b)

                    @pl.when(i > 0)
                    def _():
                        wait_scatter(b)

                    arb, bvb, obb, _, ssb = bufs[b]

                    def _unp(word):
                        lo = jax.lax.bitcast_convert_type(word << 16,
                                                          jnp.float32)
                        hi = jax.lax.bitcast_convert_type(word, jnp.float32)
                        return lo, hi

                    @plsc.parallel_loop(0, _CH, 1, unroll=4)
                    def ebody(e2):
                        for j in range(_H // 32):
                            sl = pl.ds(16 * j, 16)
                            a0, a1 = _unp(arb[e2, sl])
                            b0, b1 = _unp(bvb[e2, sl])
                            v0, v1 = _unp(bvb[e2, pl.ds(_H // 2 + 16 * j, 16)])
                            g0 = 1.0 / (1.0 + jnp.exp(-(a0 + b0)))
                            g1 = 1.0 / (1.0 + jnp.exp(-(a1 + b1)))
                            obb[e2, pl.ds(32 * j, 16)] = g0 * v0
                            obb[e2, pl.ds(32 * j + 16, 16)] = g1 * v1
                            obb[e2, pl.ds(_H + 32 * j, 16)] = g0
                            obb[e2, pl.ds(_H + 32 * j + 16, 16)] = g1

                    pltpu.async_copy(obb, accum.at[dsti.at[ck]], ssb,
                                     add=True)
                return c2
            lax.fori_loop(0, npair, pair, 0)
            wait_scatter(0)
            wait_scatter(1)
            plsc.subcore_barrier()

            pltpu.sync_copy(accum.at[pl.ds(row0, _RPT)],
                            out.at[c, tt, pl.ds(row0, _RPT)])
            return carry
        lax.fori_loop(0, tl, tt_body, 0)

    return k


def _cmb_body(acc_ref, uh_ref, res_ref, bs_ref, o_ref):
    sm = (acc_ref[0, 0].astype(jnp.float32)
          + acc_ref[1, 0].astype(jnp.float32))
    num = sm[:, :_H]
    den = sm[:, _H:]
    g = uh_ref[0] + num / (den + 1e-6) + bs_ref[...]
    o_ref[0] = jnp.where(g >= 0, g, 0.1 * g) + res_ref[0]


def _combine(acc, uh, res, bs, tl, tin):
    return pl.pallas_call(
        _cmb_body,
        grid=(tl, _NB),
        in_specs=[
            pl.BlockSpec((_NC, 1, _BN, 128), lambda t, nb: (0, t, nb, 0)),
            pl.BlockSpec((1, _BN, _H), lambda t, nb: (t, nb, 0)),
            pl.BlockSpec((1, _BN, _H), lambda t, nb: (t + (tin - tl), nb, 0)),
            pl.BlockSpec((1, _H), lambda t, nb: (0, 0)),
        ],
        out_specs=pl.BlockSpec((1, _BN, _H), lambda t, nb: (t, nb, 0)),
        out_shape=jax.ShapeDtypeStruct((tl, _N, _H), jnp.float32),
    )(acc, uh, res, bs.reshape(1, _H))


def _dec_body(acc_ref, uh_ref, res_ref, bs_ref, o_ref, x_ref, r1_ref,
              rb1_ref, r2_ref, rb2_ref, r3_ref, rb3_ref, r4_ref, rb4_ref,
              y_ref):
    sm = acc_ref[0, 0] + acc_ref[1, 0]
    g = uh_ref[0] + sm[:, :_H] / (sm[:, _H:] + 1e-6) + bs_ref[...]
    z = jnp.where(g >= 0, g, 0.1 * g) + res_ref[0] + o_ref[0]
    m = _lk(_dot(z, r1_ref[...]) + rb1_ref[...])
    m = _lk(_dot(m, r2_ref[...]) + rb2_ref[...])
    m = _lk(_dot(m, r3_ref[...]) + rb3_ref[...])
    m = _dot(m, r4_ref[...]) + rb4_ref[...]
    xb = x_ref[0]
    y_ref[...] = m + jnp.concatenate([xb, xb, xb, xb], axis=1)


def _decoder(acc, uh, res, bs, out_acc, x3, p, tl, tin):
    def wspec(shape):
        return pl.BlockSpec(shape, lambda nb: (0,) * len(shape))
    last = lambda nb: (tl - 1, nb, 0)
    return pl.pallas_call(
        _dec_body,
        grid=(_NB,),
        in_specs=[
            pl.BlockSpec((_NC, 1, _BN, 128), lambda nb: (0, tl - 1, nb, 0)),
            pl.BlockSpec((1, _BN, _H), last),
            pl.BlockSpec((1, _BN, _H), lambda nb: (tl - 1 + (tin - tl), nb, 0)),
            wspec((1, _H)),
            pl.BlockSpec((1, _BN, _H), last),
            pl.BlockSpec((1, _BN, _F), lambda nb: (_T - 1, nb, 0)),
            wspec((_H, 2 * _H)), wspec((1, 2 * _H)),
            wspec((2 * _H, 2 * _H)), wspec((1, 2 * _H)),
            wspec((2 * _H, 2 * _H)), wspec((1, 2 * _H)),
            wspec((2 * _H, _HOR * _OUTF)), wspec((1, _HOR * _OUTF)),
        ],
        out_specs=pl.BlockSpec((_BN, _HOR * _OUTF), lambda nb: (nb, 0)),
        out_shape=jax.ShapeDtypeStruct((_N, _HOR * _OUTF), jnp.float32),
    )(acc, uh, res, bs.reshape(1, _H), out_acc, x3,
      p["R1"], p["rb1"].reshape(1, -1), p["R2"], p["rb2"].reshape(1, -1),
      p["R3"], p["rb3"].reshape(1, -1), p["R4"], p["rb4"].reshape(1, -1))


def kernel(x, exog, params, edge_index):
    p = params
    x3 = x[0]
    ex = exog[0, :, :, 1:2]
    wu_eff = p["cond_Wu"][0:1] / 365.0 + p["cond_Wu"][1:2]

    h = _encoder(x3, ex, p["cond_Wx"], wu_eff, p["cond_b"].reshape(1, _H),
                 p["cond_skip"], p["enc_W1"], p["enc_b1"].reshape(1, 2 * _H),
                 p["enc_W2"], p["enc_b2"].reshape(1, _H), p["node_emb"])

    pad = _EPAD - _E
    srcp = jnp.concatenate([edge_index[0], jnp.zeros((pad,), jnp.int32)])
    dstp = jnp.concatenate([edge_index[1], jnp.zeros((pad,), jnp.int32)])
    srcp = srcp.reshape(_NW, _NCH, _CH)
    dstp = dstp.reshape(_NW, _NCH, _CH)

    out_acc = None
    tin = _T
    tprev = _T - 1
    for l in range(2):
        lp = p["layers"][l]
        d = 2 ** (l % 2)
        tl = tin - d
        ah, bvt, uh, out_acc = _temporal(h, out_acc, lp, d, tl, tprev)
        acc = _edge_sc(tl)(ah, bvt, srcp, dstp)
        if l < 1:
            h = _combine(acc, uh, h, lp["bs"], tl, tin)
            tin = tl
            tprev = tl

    y = _decoder(acc, uh, h, p["layers"][1]["bs"], out_acc, x3, p, tl, tin)
    res = y.reshape(_N, _HOR, _OUTF).transpose(1, 0, 2)[None]
    return res


# speedup vs baseline: 1.8606x; 1.0006x over previous
import functools

import jax
import jax.numpy as jnp
import numpy as np
from jax import lax
from jax.experimental import pallas as pl
from jax.experimental.pallas import tpu as pltpu
from jax.experimental.pallas import tpu_sc as plsc

_N = 10000
_E = 160000
_T = 8
_F = 26
_H = 64
_HOR = 4
_OUTF = 26

_NB = 5
_BN = _N // _NB

_NC = 2
_NS = 16
_NW = _NC * _NS
_CH = 64
_PER_TILE = 5120
_EPAD = _PER_TILE * _NW
_NCH = _PER_TILE // _CH
_NPAD = 10000
_RPT = _NPAD // _NS
_NCH_LAST = (_E - (_NW - 1) * _PER_TILE) // _CH

_PLO = np.array([32 * g + k for g in range(_H // 32) for k in range(16)],
                np.int32)
_PHI = _PLO + 16


def _lk(v, s=0.01):
    return jnp.where(v >= 0, v, s * v)


def _dot(a, b):
    return jnp.dot(a, b, preferred_element_type=jnp.float32)


def _enc_body(x_ref, e_ref, wx_ref, wu_ref, b_ref, wsk_ref, w1_ref, b1_ref,
              w2_ref, b2_ref, emb_ref, o_ref):
    xb = x_ref[0]
    eb = e_ref[0]
    h = _lk(_dot(xb, wx_ref[...]) + eb * wu_ref[...] + b_ref[...])
    h = h + _dot(xb, wsk_ref[...])
    h = _lk(_dot(h, w1_ref[...]) + b1_ref[...])
    h = _lk(_dot(h, w2_ref[...]) + b2_ref[...])
    o_ref[0] = h + emb_ref[...]


def _encoder(x3, ex, wx, wu_eff, b, wsk, w1, b1, w2, b2, emb):
    def wspec(shape):
        return pl.BlockSpec(shape, lambda t, nb: (0,) * len(shape))
    return pl.pallas_call(
        _enc_body,
        grid=(_T, _NB),
        in_specs=[
            pl.BlockSpec((1, _BN, _F), lambda t, nb: (t, nb, 0)),
            pl.BlockSpec((1, _BN, 1), lambda t, nb: (t, nb, 0)),
            wspec((_F, _H)), wspec((1, _H)), wspec((1, _H)), wspec((_F, _H)),
            wspec((_H, 2 * _H)), wspec((1, 2 * _H)),
            wspec((2 * _H, _H)), wspec((1, _H)),
            pl.BlockSpec((_BN, _H), lambda t, nb: (nb, 0)),
        ],
        out_specs=pl.BlockSpec((1, _BN, _H), lambda t, nb: (t, nb, 0)),
        out_shape=jax.ShapeDtypeStruct((_T, _N, _H), jnp.float32),
    )(x3, ex, wx, wu_eff, b, wsk, w1, b1, w2, b2, emb)


def _bfpack(xlo, xhi):
    ulo = jax.lax.bitcast_convert_type(xlo, jnp.uint32)
    uhi = jax.lax.bitcast_convert_type(xhi, jnp.uint32)
    one = jnp.uint32(1)
    half = jnp.uint32(0x7FFF)
    rlo = (ulo + half + ((ulo >> 16) & one)) >> 16
    rhi = (uhi + half + ((uhi >> 16) & one)) >> 16
    return jax.lax.bitcast_convert_type(rlo | (rhi << 16), jnp.int32)


def _tmp_body_mk(has_prev):
    def body(*refs):
        if has_prev:
            (h1_ref, h0_ref, po_ref, wt1_ref, wt0_ref, bt_ref, walo_ref,
             wahi_ref, wblo_ref, wbhi_ref, wvlo_ref, wvhi_ref, wu_ref,
             ws_ref, bsk_ref, ah_ref, bv_ref, uh_ref, on_ref) = refs
        else:
            (h1_ref, h0_ref, wt1_ref, wt0_ref, bt_ref, walo_ref,
             wahi_ref, wblo_ref, wbhi_ref, wvlo_ref, wvhi_ref, wu_ref,
             ws_ref, bsk_ref, ah_ref, bv_ref, uh_ref, on_ref) = refs
        hc = _lk(_dot(h1_ref[0], wt1_ref[...]) + _dot(h0_ref[0], wt0_ref[...])
                 + bt_ref[...])
        ah_ref[...] = _bfpack(_dot(hc, walo_ref[...]), _dot(hc, wahi_ref[...]))
        bv_ref[...] = jnp.concatenate(
            [_bfpack(_dot(hc, wblo_ref[...]), _dot(hc, wbhi_ref[...])),
             _bfpack(_dot(hc, wvlo_ref[...]), _dot(hc, wvhi_ref[...]))],
            axis=1)
        uh_ref[0] = _dot(hc, wu_ref[...])
        on = _dot(hc, ws_ref[...]) + bsk_ref[...]
        if has_prev:
            on = on + po_ref[0]
        on_ref[0] = on
    return body


def _temporal(h, po, lp, d, tl, tprev):
    def wspec(shape):
        return pl.BlockSpec(shape, lambda t, nb: (0,) * len(shape))
    hspec = lambda off: pl.BlockSpec((1, _BN, _H), lambda t, nb: (t + off, nb, 0))
    tab_spec = lambda wdt: pl.BlockSpec((_BN, wdt), lambda t, nb: (t * _NB + nb, 0))
    tab_shape = lambda wdt: jax.ShapeDtypeStruct((tl * _N + 16, wdt), jnp.int32)
    seq_spec = pl.BlockSpec((1, _BN, _H), lambda t, nb: (t, nb, 0))
    seq_shape = jax.ShapeDtypeStruct((tl, _N, _H), jnp.float32)
    hw = _H // 2
    has_prev = po is not None
    po_spec = ([pl.BlockSpec((1, _BN, _H),
                             lambda t, nb: (t + (tprev - tl), nb, 0))]
               if has_prev else [])
    po_arg = [po] if has_prev else []
    return pl.pallas_call(
        _tmp_body_mk(has_prev),
        grid=(tl, _NB),
        in_specs=[
            hspec(d), hspec(0), *po_spec,
            wspec((_H, _H)), wspec((_H, _H)), wspec((1, _H)),
            wspec((_H, hw)), wspec((_H, hw)), wspec((_H, hw)),
            wspec((_H, hw)), wspec((_H, hw)), wspec((_H, hw)),
            wspec((_H, _H)), wspec((_H, _H)), wspec((1, _H)),
        ],
        out_specs=[tab_spec(hw), tab_spec(_H), seq_spec, seq_spec],
        out_shape=[tab_shape(hw), tab_shape(_H), seq_shape, seq_shape],
    )(h, h, *po_arg, lp["Wt1"], lp["Wt0"], lp["bt"].reshape(1, _H),
      lp["A"][:, _PLO], lp["A"][:, _PHI], lp["B"][:, _PLO], lp["B"][:, _PHI],
      lp["V"][:, _PLO], lp["V"][:, _PHI], lp["U"], lp["Ws"],
      lp["bskip"].reshape(1, _H))


def _edge_sc(tl):
    mesh = plsc.VectorSubcoreMesh(core_axis_name="c", subcore_axis_name="s")

    @functools.partial(
        pl.kernel,
        out_type=jax.ShapeDtypeStruct((_NC, tl, _NPAD, 128), jnp.float32),
        mesh=mesh,
        compiler_params=pltpu.CompilerParams(use_tc_tiling_on_sc=False),
        scratch_types=[
            pltpu.VMEM((_NCH, _CH), jnp.int32),
            pltpu.VMEM((_NCH, _CH), jnp.int32),
            pltpu.VMEM((_NCH, _CH), jnp.int32),
            pltpu.VMEM((_CH, _H // 2), jnp.int32),
            pltpu.VMEM((_CH, _H), jnp.int32),
            pltpu.VMEM((_CH, _H // 2), jnp.int32),
            pltpu.VMEM((_CH, _H), jnp.int32),
            pltpu.VMEM((_CH, 128), jnp.float32),
            pltpu.VMEM((_CH, 128), jnp.float32),
            pltpu.VMEM((32, 128), jnp.float32),
            pltpu.VMEM_SHARED((_NPAD, 128), jnp.float32),
            pltpu.SemaphoreType.DMA,
            pltpu.SemaphoreType.DMA,
            pltpu.SemaphoreType.DMA,
            pltpu.SemaphoreType.DMA,
            pltpu.SemaphoreType.DMA,
        ],
    )
    def k(ah, bv, srcr, dstr, out, srco, dsti, dsto, ar0, bv0,
          ar1, bv1, ob0, ob1, zbuf, accum, sg0, sg1, ss0, ss1, sz):
        c = lax.axis_index("c")
        s = lax.axis_index("s")
        w = c * _NS + s
        row0 = s * _RPT
        npair = jnp.where(w == _NW - 1, _NCH_LAST // 2, _NCH // 2)
        bufs = ((ar0, bv0, ob0, sg0, ss0),
                (ar1, bv1, ob1, sg1, ss1))

        rbase = w * _NCH

        @pl.when(w < _NW - 1)
        def _():
            pltpu.sync_copy(srcr.at[pl.ds(rbase, _NCH)], srco)
            pltpu.sync_copy(dstr.at[pl.ds(rbase, _NCH)], dsti)
            pltpu.sync_copy(dstr.at[pl.ds(rbase, _NCH)], dsto)

        @pl.when(w == _NW - 1)
        def _():
            pltpu.sync_copy(srcr.at[pl.ds(rbase, _NCH_LAST)],
                            srco.at[pl.ds(0, _NCH_LAST)])
            pltpu.sync_copy(dstr.at[pl.ds(rbase, _NCH_LAST)],
                            dsti.at[pl.ds(0, _NCH_LAST)])
            pltpu.sync_copy(dstr.at[pl.ds(rbase, _NCH_LAST)],
                            dsto.at[pl.ds(0, _NCH_LAST)])

        def zrow(r, carry):
            for j in range(8):
                zbuf[r, pl.ds(16 * j, 16)] = jnp.zeros((16,), jnp.float32)
            return carry
        lax.fori_loop(0, 32, zrow, 0)

        def issue_gather(ck, b):
            arb, bvb, sgb = bufs[b][0], bufs[b][1], bufs[b][3]
            pltpu.async_copy(ah.at[dsto.at[ck]], arb, sgb)
            pltpu.async_copy(bv.at[srco.at[ck]], bvb, sgb)

        def wait_gather(b):
            arb, bvb, sgb = bufs[b][0], bufs[b][1], bufs[b][3]
            pltpu.make_async_copy(ah.at[dsto.at[0]], arb, sgb).wait()
            pltpu.make_async_copy(bv.at[srco.at[0]], bvb, sgb).wait()

        def wait_scatter(b):
            obb, ssb = bufs[b][2], bufs[b][4]
            pltpu.make_async_copy(obb, accum.at[dsti.at[0]], ssb).wait()

        def tt_body(tt, carry):
            zrem = _RPT % 32

            def zcp(kk, cc):
                pltpu.async_copy(zbuf, accum.at[pl.ds(row0 + kk * 32, 32)],
                                 sz)
                return cc
            lax.fori_loop(0, _RPT // 32, zcp, 0)
            if zrem:
                pltpu.async_copy(zbuf.at[pl.ds(0, zrem)],
                                 accum.at[pl.ds(row0 + _RPT - zrem, zrem)],
                                 sz)

            def zwt(kk, cc):
                pltpu.make_async_copy(zbuf,
                                      accum.at[pl.ds(row0, 32)], sz).wait()
                return cc
            lax.fori_loop(0, _RPT // 32, zwt, 0)
            if zrem:
                pltpu.make_async_copy(zbuf.at[pl.ds(0, zrem)],
                                      accum.at[pl.ds(row0, zrem)], sz).wait()
            plsc.subcore_barrier()

            def off(ck2, c2):
                for j in range(_CH // 16):
                    sl = pl.ds(16 * j, 16)
                    srco[ck2, sl] = srco[ck2, sl] + _N
                    dsto[ck2, sl] = dsto[ck2, sl] + _N
                return c2

            @pl.when(tt > 0)
            def _():
                lax.fori_loop(0, _NCH, off, 0)

            issue_gather(0, 0)

            def pair(i, c2):
                for b in (0, 1):
                    ck = 2 * i + b
                    if b == 0:
                        issue_gather(ck + 1, 1)
                    else:
                        @pl.when(i < npair - 1)
                        def _():
                            issue_gather(ck + 1, 0)
                    wait_gather(b)

                    @pl.when(i > 0)
                    def _():
                        wait_scatter(b)

                    arb, bvb, obb, _, ssb = bufs[b]

                    def _unp(word):
                        lo = jax.lax.bitcast_convert_type(word << 16,
                                                          jnp.float32)
                        hi = jax.lax.bitcast_convert_type(word, jnp.float32)
                        return lo, hi

                    @plsc.parallel_loop(0, _CH, 1, unroll=4)
                    def ebody(e2):
                        for j in range(_H // 32):
                            sl = pl.ds(16 * j, 16)
                            a0, a1 = _unp(arb[e2, sl])
                            b0, b1 = _unp(bvb[e2, sl])
                            v0, v1 = _unp(bvb[e2, pl.ds(_H // 2 + 16 * j, 16)])
                            g0 = 1.0 / (1.0 + jnp.exp(-(a0 + b0)))
                            g1 = 1.0 / (1.0 + jnp.exp(-(a1 + b1)))
                            obb[e2, pl.ds(32 * j, 16)] = g0 * v0
                            obb[e2, pl.ds(32 * j + 16, 16)] = g1 * v1
                            obb[e2, pl.ds(_H + 32 * j, 16)] = g0
                            obb[e2, pl.ds(_H + 32 * j + 16, 16)] = g1

                    pltpu.async_copy(obb, accum.at[dsti.at[ck]], ssb,
                                     add=True)
                return c2
            lax.fori_loop(0, npair, pair, 0)
            wait_scatter(0)
            wait_scatter(1)
            plsc.subcore_barrier()

            pltpu.sync_copy(accum.at[pl.ds(row0, _RPT)],
                            out.at[c, tt, pl.ds(row0, _RPT)])
            return carry
        lax.fori_loop(0, tl, tt_body, 0)

    return k


def _cmb_body(acc_ref, uh_ref, res_ref, bs_ref, o_ref):
    sm = (acc_ref[0, 0].astype(jnp.float32)
          + acc_ref[1, 0].astype(jnp.float32))
    num = sm[:, :_H]
    den = sm[:, _H:]
    g = uh_ref[0] + num / (den + 1e-6) + bs_ref[...]
    o_ref[0] = jnp.where(g >= 0, g, 0.1 * g) + res_ref[0]


def _combine(acc, uh, res, bs, tl, tin):
    return pl.pallas_call(
        _cmb_body,
        grid=(tl, _NB),
        in_specs=[
            pl.BlockSpec((_NC, 1, _BN, 128), lambda t, nb: (0, t, nb, 0)),
            pl.BlockSpec((1, _BN, _H), lambda t, nb: (t, nb, 0)),
            pl.BlockSpec((1, _BN, _H), lambda t, nb: (t + (tin - tl), nb, 0)),
            pl.BlockSpec((1, _H), lambda t, nb: (0, 0)),
        ],
        out_specs=pl.BlockSpec((1, _BN, _H), lambda t, nb: (t, nb, 0)),
        out_shape=jax.ShapeDtypeStruct((tl, _N, _H), jnp.float32),
    )(acc, uh, res, bs.reshape(1, _H))


def _dec_body(acc_ref, uh_ref, res_ref, bs_ref, o_ref, x_ref, r1_ref,
              rb1_ref, r2_ref, rb2_ref, r3_ref, rb3_ref, r4_ref, rb4_ref,
              y_ref):
    sm = acc_ref[0, 0] + acc_ref[1, 0]
    g = uh_ref[0] + sm[:, :_H] / (sm[:, _H:] + 1e-6) + bs_ref[...]
    z = jnp.where(g >= 0, g, 0.1 * g) + res_ref[0] + o_ref[0]
    m = _lk(_dot(z, r1_ref[...]) + rb1_ref[...])
    m = _lk(_dot(m, r2_ref[...]) + rb2_ref[...])
    m = _lk(_dot(m, r3_ref[...]) + rb3_ref[...])
    m = _dot(m, r4_ref[...]) + rb4_ref[...]
    xb = x_ref[0]
    y_ref[...] = m + jnp.concatenate([xb, xb, xb, xb], axis=1)


def _decoder(acc, uh, res, bs, out_acc, x3, p, tl, tin):
    def wspec(shape):
        return pl.BlockSpec(shape, lambda nb: (0,) * len(shape))
    last = lambda nb: (tl - 1, nb, 0)
    return pl.pallas_call(
        _dec_body,
        grid=(_NB,),
        in_specs=[
            pl.BlockSpec((_NC, 1, _BN, 128), lambda nb: (0, tl - 1, nb, 0)),
            pl.BlockSpec((1, _BN, _H), last),
            pl.BlockSpec((1, _BN, _H), lambda nb: (tl - 1 + (tin - tl), nb, 0)),
            wspec((1, _H)),
            pl.BlockSpec((1, _BN, _H), last),
            pl.BlockSpec((1, _BN, _F), lambda nb: (_T - 1, nb, 0)),
            wspec((_H, 2 * _H)), wspec((1, 2 * _H)),
            wspec((2 * _H, 2 * _H)), wspec((1, 2 * _H)),
            wspec((2 * _H, 2 * _H)), wspec((1, 2 * _H)),
            wspec((2 * _H, _HOR * _OUTF)), wspec((1, _HOR * _OUTF)),
        ],
        out_specs=pl.BlockSpec((_BN, _HOR * _OUTF), lambda nb: (nb, 0)),
        out_shape=jax.ShapeDtypeStruct((_N, _HOR * _OUTF), jnp.float32),
    )(acc, uh, res, bs.reshape(1, _H), out_acc, x3,
      p["R1"], p["rb1"].reshape(1, -1), p["R2"], p["rb2"].reshape(1, -1),
      p["R3"], p["rb3"].reshape(1, -1), p["R4"], p["rb4"].reshape(1, -1))


def kernel(x, exog, params, edge_index):
    p = params
    x3 = x[0]
    ex = exog[0, :, :, 1:2]
    wu_eff = p["cond_Wu"][0:1] / 365.0 + p["cond_Wu"][1:2]

    h = _encoder(x3, ex, p["cond_Wx"], wu_eff, p["cond_b"].reshape(1, _H),
                 p["cond_skip"], p["enc_W1"], p["enc_b1"].reshape(1, 2 * _H),
                 p["enc_W2"], p["enc_b2"].reshape(1, _H), p["node_emb"])

    srcp = edge_index[0].reshape(_E // _CH, _CH)
    dstp = edge_index[1].reshape(_E // _CH, _CH)

    out_acc = None
    tin = _T
    tprev = _T - 1
    for l in range(2):
        lp = p["layers"][l]
        d = 2 ** (l % 2)
        tl = tin - d
        ah, bvt, uh, out_acc = _temporal(h, out_acc, lp, d, tl, tprev)
        acc = _edge_sc(tl)(ah, bvt, srcp, dstp)
        if l < 1:
            h = _combine(acc, uh, h, lp["bs"], tl, tin)
            tin = tl
            tprev = tl

    y = _decoder(acc, uh, h, p["layers"][1]["bs"], out_acc, x3, p, tl, tin)
    res = y.reshape(_N, _HOR, _OUTF).transpose(1, 0, 2)[None]
    return res
